# Initial kernel scaffold; baseline (speedup 1.0000x reference)
import jax
import jax.numpy as jnp
from jax.experimental import pallas as pl


def kernel(x, edge_index, batch, W_gat, att_src, att_dst, b_gat, W1, b1, W2, b2):
    raise NotImplementedError("write your pallas kernel here")


# speedup vs baseline: 11.5063x; 11.5063x over previous
import functools
import jax
import jax.numpy as jnp
from jax import lax
from jax.experimental import pallas as pl
from jax.experimental.pallas import tpu as pltpu
from jax.experimental.pallas import tpu_sc as plsc

N = 10000
NPAD = 10240
E = 160000
HID = 256
F = 32
G = 64
NW = 32
EPT = 5120
EPAD = NW * EPT
EPC = EPAD // 2
CE = 8192
NCH = EPC // CE
BT = 512
NB = NPAD // BT


def _k1_body(x_ref, wgt_ref, att_ref, ht_ref, avec_ref):
    ht = lax.dot_general(wgt_ref[...], x_ref[...],
                         (((1,), (1,)), ((), ())),
                         preferred_element_type=jnp.float32)
    ht_ref[:, 0:N] = ht
    ht_ref[:, N:NPAD] = jnp.zeros((F, NPAD - N), jnp.float32)
    a_s = jnp.sum(ht * att_ref[0, 0:F][:, None], axis=0)
    a_d = jnp.sum(ht * att_ref[1, 0:F][:, None], axis=0)
    e = a_s + a_d
    e = jnp.where(e >= 0, e, 0.2 * e)
    p_self = jnp.exp(e)
    zpad = jnp.zeros((NPAD - N,), jnp.float32)
    avec_ref[0, 0:N] = a_s
    avec_ref[0, N:NPAD] = zpad
    avec_ref[1, 0:N] = a_d
    avec_ref[1, N:NPAD] = jnp.full((NPAD - N,), -1e30, jnp.float32)
    avec_ref[2, 0:N] = p_self
    avec_ref[2, N:NPAD] = zpad


def _sc_edges(src_hbm, dst_hbm, avec_hbm, ht_hbm,
              s_out, o_out, p_out,
              as_v, ad_v, srcv, dstv, pv, s_loc, acc1,
              srcc, dstc, pc):
    c = lax.axis_index("c")
    s = lax.axis_index("s")
    wid = c * 16 + s
    pltpu.sync_copy(avec_hbm.at[0], as_v)
    pltpu.sync_copy(avec_hbm.at[1], ad_v)
    pltpu.sync_copy(src_hbm.at[c, pl.ds(s * EPT, EPT)], srcv)
    pltpu.sync_copy(dst_hbm.at[c, pl.ds(s * EPT, EPT)], dstv)

    zero16 = jnp.zeros((16,), jnp.float32)

    def zloop(i, _):
        s_loc[pl.ds(i * 16, 16)] = zero16
        return 0
    lax.fori_loop(0, NPAD // 16, zloop, 0)

    def ploop(t, _):
        off = t * 16
        sv = srcv[pl.ds(off, 16)]
        dv = dstv[pl.ds(off, 16)]
        a1 = plsc.load_gather(as_v, [sv])
        a2 = plsc.load_gather(ad_v, [dv])
        e = a1 + a2
        e = jnp.where(e >= 0, e, 0.2 * e)
        pe = jnp.exp(e)
        pv[pl.ds(off, 16)] = pe
        plsc.addupdate_scatter(s_loc, [dv], pe)
        return 0
    lax.fori_loop(0, EPT // 16, ploop, 0)

    pltpu.sync_copy(pv, p_out.at[c, pl.ds(s * EPT, EPT)])
    pltpu.sync_copy(s_loc, s_out.at[wid])
    plsc.subcore_barrier()

    f0 = 2 * s
    pltpu.sync_copy(ht_hbm.at[f0], as_v)
    pltpu.sync_copy(ht_hbm.at[f0 + 1], ad_v)
    lax.fori_loop(0, NPAD // 16, zloop, 0)

    def zloop1(i, _):
        acc1[pl.ds(i * 16, 16)] = zero16
        return 0
    lax.fori_loop(0, NPAD // 16, zloop1, 0)

    def chunk(k, _):
        pltpu.sync_copy(src_hbm.at[c, pl.ds(k * CE, CE)], srcc)
        pltpu.sync_copy(dst_hbm.at[c, pl.ds(k * CE, CE)], dstc)
        pltpu.sync_copy(p_out.at[c, pl.ds(k * CE, CE)], pc)

        def inner(i, _):
            off = i * 16
            sv = srcc[pl.ds(off, 16)]
            dv = dstc[pl.ds(off, 16)]
            pe = pc[pl.ds(off, 16)]
            g0 = plsc.load_gather(as_v, [sv])
            plsc.addupdate_scatter(s_loc, [dv], g0 * pe)
            g1 = plsc.load_gather(ad_v, [sv])
            plsc.addupdate_scatter(acc1, [dv], g1 * pe)
            return 0
        lax.fori_loop(0, CE // 16, inner, 0)
        return 0
    lax.fori_loop(0, NCH, chunk, 0)

    pltpu.sync_copy(s_loc, o_out.at[c, f0])
    pltpu.sync_copy(acc1, o_out.at[c, f0 + 1])


def _run_sc(srcr, dstr, avec, ht):
    mesh = plsc.VectorSubcoreMesh(core_axis_name="c", subcore_axis_name="s",
                                  num_cores=2, num_subcores=16)
    f = functools.partial(
        pl.kernel,
        out_type=[jax.ShapeDtypeStruct((NW, NPAD), jnp.float32),
                  jax.ShapeDtypeStruct((2, F, NPAD), jnp.float32),
                  jax.ShapeDtypeStruct((2, EPC), jnp.float32)],
        mesh=mesh,
        compiler_params=pltpu.CompilerParams(needs_layout_passes=False),
        scratch_types=[
            pltpu.VMEM((NPAD,), jnp.float32),
            pltpu.VMEM((NPAD,), jnp.float32),
            pltpu.VMEM((EPT,), jnp.int32),
            pltpu.VMEM((EPT,), jnp.int32),
            pltpu.VMEM((EPT,), jnp.float32),
            pltpu.VMEM((NPAD,), jnp.float32),
            pltpu.VMEM((NPAD,), jnp.float32),
            pltpu.VMEM((CE,), jnp.int32),
            pltpu.VMEM((CE,), jnp.int32),
            pltpu.VMEM((CE,), jnp.float32),
        ],
    )(_sc_edges)
    return f(srcr, dstr, avec, ht)


def _k3_body(opart_ref, spart_ref, avec_ref, ht_ref, w2t_ref, wb_ref,
             bpm_ref, x1_ref, x64_ref):
    num = opart_ref[0] + opart_ref[1]
    s_edges = jnp.sum(spart_ref[...], axis=0)
    p_self = avec_ref[2, :]
    ht = ht_ref[...]
    s_tot = s_edges + p_self + 1e-16
    xc = (num + p_self[None, :] * ht) / s_tot[None, :]
    xc = xc + wb_ref[0, 0:F][:, None]
    bpm_ref[...] = jnp.where(xc > 0, 1.0, -1.0)
    x1_ref[...] = jnp.sum(xc * wb_ref[1, 0:F][:, None], axis=0) + wb_ref[3, 0]
    x64 = lax.dot_general(w2t_ref[...], xc, (((1,), (0,)), ((), ())),
                          preferred_element_type=jnp.float32)
    x64_ref[...] = x64 + wb_ref[2, 0:G][:, None]


def _k4_body(bpm_i_ref, bpm_j_ref, bat_i_ref, bat_j_ref, dup_ref):
    i = pl.program_id(0)
    j = pl.program_id(1)

    @pl.when(j == 0)
    def _():
        dup_ref[...] = jnp.zeros((BT,), jnp.float32)

    @pl.when(j <= i)
    def _():
        sim = lax.dot_general(bpm_i_ref[...], bpm_j_ref[...],
                              (((0,), (0,)), ((), ())),
                              preferred_element_type=jnp.float32)
        eq = sim > 31.5
        bm = bat_i_ref[...][:, None] == bat_j_ref[...][None, :]
        gi = i * BT + lax.broadcasted_iota(jnp.int32, (BT, BT), 0)
        gj = j * BT + lax.broadcasted_iota(jnp.int32, (BT, BT), 1)
        lt = gj < gi
        anyj = jnp.max(jnp.where(eq & bm & lt, 1.0, 0.0), axis=1)
        dup_ref[...] = jnp.maximum(dup_ref[...], anyj)


def _k5_body(x1_ref, x64_ref, dup_ref, bat_ref, gx_ref):
    x1 = x1_ref[...]
    pw = jnp.where(dup_ref[...] < 0.5, jnp.exp(x1), 0.0)
    bat = bat_ref[...]
    gids = lax.broadcasted_iota(jnp.int32, (G, NPAD), 0).astype(jnp.float32)
    oht = jnp.where(gids == bat[None, :], 1.0, 0.0)
    wx = x64_ref[...] * pw[None, :]
    numm = lax.dot_general(oht, wx, (((1,), (1,)), ((), ())),
                           preferred_element_type=jnp.float32)
    den = jnp.sum(oht * pw[None, :], axis=1, keepdims=True)
    gx_ref[...] = numm / (den + 1e-16)


def kernel(x, edge_index, batch, W_gat, att_src, att_dst, b_gat,
           W1, b1, W2, b2):
    src = edge_index[0]
    dst = edge_index[1]
    srcr = jnp.concatenate(
        [src, jnp.zeros((EPAD - E,), jnp.int32)]).reshape(2, EPC)
    dstr = jnp.concatenate(
        [dst, jnp.full((EPAD - E,), N, jnp.int32)]).reshape(2, EPC)
    wgt_pad = jnp.zeros((F, HID), jnp.float32).at[:30, :].set(W_gat.T)
    att2 = (jnp.zeros((2, F), jnp.float32)
            .at[0, :30].set(att_src).at[1, :30].set(att_dst))
    wb = (jnp.zeros((4, 128), jnp.float32)
          .at[0, :30].set(b_gat)
          .at[1, :30].set(W1[:, 0])
          .at[2, :G].set(b2)
          .at[3, 0].set(b1[0]))
    w2t_pad = jnp.zeros((G, F), jnp.float32).at[:, :30].set(W2.T)
    bat_f = jnp.concatenate(
        [batch.astype(jnp.float32), jnp.full((NPAD - N,), 1e9, jnp.float32)])

    ht, avec = pl.pallas_call(
        _k1_body,
        out_shape=[jax.ShapeDtypeStruct((F, NPAD), jnp.float32),
                   jax.ShapeDtypeStruct((8, NPAD), jnp.float32)],
    )(x, wgt_pad, att2)

    s_part, o_part, _ = _run_sc(srcr, dstr, avec, ht)

    bpm, x1, x64 = pl.pallas_call(
        _k3_body,
        out_shape=[jax.ShapeDtypeStruct((F, NPAD), jnp.float32),
                   jax.ShapeDtypeStruct((NPAD,), jnp.float32),
                   jax.ShapeDtypeStruct((G, NPAD), jnp.float32)],
    )(o_part, s_part, avec, ht, w2t_pad, wb)

    dup = pl.pallas_call(
        _k4_body,
        grid=(NB, NB),
        in_specs=[
            pl.BlockSpec((F, BT), lambda i, j: (0, i)),
            pl.BlockSpec((F, BT), lambda i, j: (0, j)),
            pl.BlockSpec((BT,), lambda i, j: (i,)),
            pl.BlockSpec((BT,), lambda i, j: (j,)),
        ],
        out_specs=pl.BlockSpec((BT,), lambda i, j: (i,)),
        out_shape=jax.ShapeDtypeStruct((NPAD,), jnp.float32),
    )(bpm, bpm, bat_f, bat_f)

    gx = pl.pallas_call(
        _k5_body,
        out_shape=jax.ShapeDtypeStruct((G, G), jnp.float32),
    )(x1, x64, dup, bat_f)
    return gx


# speedup vs baseline: 19.0401x; 1.6547x over previous
import functools
import jax
import jax.numpy as jnp
from jax import lax
from jax.experimental import pallas as pl
from jax.experimental.pallas import tpu as pltpu
from jax.experimental.pallas import tpu_sc as plsc

N = 10000
NPAD = 10240
E = 160000
HID = 256
F = 32
G = 64
NW = 32
EPT = 5120
EPAD = NW * EPT
EPC = EPAD // 2
CE = 4096
NCH = EPC // CE
BT = 1024
NB = NPAD // BT
LAM = 45.254834
ANG = 0.09817477


def _k1_body(x_ref, wgt_ref, att_ref, ht_ref, avec_ref):
    ht = lax.dot_general(wgt_ref[...], x_ref[...],
                         (((1,), (1,)), ((), ())),
                         preferred_element_type=jnp.float32)
    ht_ref[:, 0:N] = ht
    ht_ref[:, N:NPAD] = jnp.zeros((F, NPAD - N), jnp.float32)
    a_s = jnp.sum(ht * att_ref[0, 0:F][:, None], axis=0)
    a_d = jnp.sum(ht * att_ref[1, 0:F][:, None], axis=0)
    e = a_s + a_d
    e = jnp.where(e >= 0, e, 0.2 * e)
    p_self = jnp.exp(e)
    zpad = jnp.zeros((NPAD - N,), jnp.float32)
    avec_ref[0, 0:N] = a_s
    avec_ref[0, N:NPAD] = zpad
    avec_ref[1, 0:N] = a_d
    avec_ref[1, N:NPAD] = jnp.full((NPAD - N,), -1e30, jnp.float32)
    avec_ref[2, 0:N] = p_self
    avec_ref[2, N:NPAD] = zpad


def _sc_edges(src_hbm, dst_hbm, avec_hbm, ht_hbm,
              s_out, o_out, p_out,
              as_v, ad_v, srcv, dstv, pv, s_loc, acc1,
              srcc, dstc, pc, srcc2, dstc2, pc2, sem0, sema, semb):
    c = lax.axis_index("c")
    s = lax.axis_index("s")
    wid = c * 16 + s
    pltpu.async_copy(avec_hbm.at[0], as_v, sem0)
    pltpu.async_copy(avec_hbm.at[1], ad_v, sem0)
    pltpu.async_copy(src_hbm.at[c, pl.ds(s * EPT, EPT)], srcv, sem0)
    pltpu.async_copy(dst_hbm.at[c, pl.ds(s * EPT, EPT)], dstv, sem0)

    zero16 = jnp.zeros((16,), jnp.float32)

    def zloop(i, _):
        for u in range(4):
            s_loc[pl.ds(i * 64 + u * 16, 16)] = zero16
        return 0
    lax.fori_loop(0, NPAD // 64, zloop, 0)

    pltpu.make_async_copy(avec_hbm.at[0], as_v, sem0).wait()
    pltpu.make_async_copy(avec_hbm.at[1], ad_v, sem0).wait()
    pltpu.make_async_copy(src_hbm.at[c, pl.ds(s * EPT, EPT)], srcv, sem0).wait()
    pltpu.make_async_copy(dst_hbm.at[c, pl.ds(s * EPT, EPT)], dstv, sem0).wait()

    def ploop(t, _):
        for u in range(2):
            off = t * 32 + u * 16
            sv = srcv[pl.ds(off, 16)]
            dv = dstv[pl.ds(off, 16)]
            a1 = plsc.load_gather(as_v, [sv])
            a2 = plsc.load_gather(ad_v, [dv])
            e = a1 + a2
            e = jnp.where(e >= 0, e, 0.2 * e)
            pe = jnp.exp(e)
            pv[pl.ds(off, 16)] = pe
            plsc.addupdate_scatter(s_loc, [dv], pe)
        return 0
    lax.fori_loop(0, EPT // 32, ploop, 0)

    pltpu.sync_copy(pv, p_out.at[c, pl.ds(s * EPT, EPT)])
    pltpu.sync_copy(s_loc, s_out.at[wid])
    plsc.subcore_barrier()

    f0 = 2 * s
    pltpu.async_copy(ht_hbm.at[f0], as_v, sem0)
    pltpu.async_copy(ht_hbm.at[f0 + 1], ad_v, sem0)

    def zloop2(i, _):
        for u in range(4):
            s_loc[pl.ds(i * 64 + u * 16, 16)] = zero16
            acc1[pl.ds(i * 64 + u * 16, 16)] = zero16
        return 0
    lax.fori_loop(0, NPAD // 64, zloop2, 0)

    pltpu.make_async_copy(ht_hbm.at[f0], as_v, sem0).wait()
    pltpu.make_async_copy(ht_hbm.at[f0 + 1], ad_v, sem0).wait()

    def fire(k, sb, db, pb, sem):
        pltpu.async_copy(src_hbm.at[c, pl.ds(k * CE, CE)], sb, sem)
        pltpu.async_copy(dst_hbm.at[c, pl.ds(k * CE, CE)], db, sem)
        pltpu.async_copy(p_out.at[c, pl.ds(k * CE, CE)], pb, sem)

    def drain(k, sb, db, pb, sem):
        pltpu.make_async_copy(src_hbm.at[c, pl.ds(k * CE, CE)], sb, sem).wait()
        pltpu.make_async_copy(dst_hbm.at[c, pl.ds(k * CE, CE)], db, sem).wait()
        pltpu.make_async_copy(p_out.at[c, pl.ds(k * CE, CE)], pb, sem).wait()

    def compute(sb, db, pb):
        def inner(i, _):
            for u in range(2):
                off = i * 32 + u * 16
                sv = sb[pl.ds(off, 16)]
                dv = db[pl.ds(off, 16)]
                pe = pb[pl.ds(off, 16)]
                g0 = plsc.load_gather(as_v, [sv])
                plsc.addupdate_scatter(s_loc, [dv], g0 * pe)
                g1 = plsc.load_gather(ad_v, [sv])
                plsc.addupdate_scatter(acc1, [dv], g1 * pe)
            return 0
        lax.fori_loop(0, CE // 32, inner, 0)

    fire(0, srcc, dstc, pc, sema)

    def chunk2(g, _):
        k0 = g * 2
        fire(k0 + 1, srcc2, dstc2, pc2, semb)
        drain(k0, srcc, dstc, pc, sema)
        compute(srcc, dstc, pc)

        @pl.when(k0 + 2 < NCH)
        def _():
            fire(k0 + 2, srcc, dstc, pc, sema)
        drain(k0 + 1, srcc2, dstc2, pc2, semb)
        compute(srcc2, dstc2, pc2)
        return 0
    lax.fori_loop(0, NCH // 2, chunk2, 0)

    pltpu.sync_copy(s_loc, o_out.at[c, f0])
    pltpu.sync_copy(acc1, o_out.at[c, f0 + 1])


def _run_sc(srcr, dstr, avec, ht):
    mesh = plsc.VectorSubcoreMesh(core_axis_name="c", subcore_axis_name="s",
                                  num_cores=2, num_subcores=16)
    f = functools.partial(
        pl.kernel,
        out_type=[jax.ShapeDtypeStruct((NW, NPAD), jnp.float32),
                  jax.ShapeDtypeStruct((2, F, NPAD), jnp.float32),
                  jax.ShapeDtypeStruct((2, EPC), jnp.float32)],
        mesh=mesh,
        compiler_params=pltpu.CompilerParams(needs_layout_passes=False),
        scratch_types=[
            pltpu.VMEM((NPAD,), jnp.float32),
            pltpu.VMEM((NPAD,), jnp.float32),
            pltpu.VMEM((EPT,), jnp.int32),
            pltpu.VMEM((EPT,), jnp.int32),
            pltpu.VMEM((EPT,), jnp.float32),
            pltpu.VMEM((NPAD,), jnp.float32),
            pltpu.VMEM((NPAD,), jnp.float32),
            pltpu.VMEM((CE,), jnp.int32),
            pltpu.VMEM((CE,), jnp.int32),
            pltpu.VMEM((CE,), jnp.float32),
            pltpu.VMEM((CE,), jnp.int32),
            pltpu.VMEM((CE,), jnp.int32),
            pltpu.VMEM((CE,), jnp.float32),
            pltpu.SemaphoreType.DMA,
            pltpu.SemaphoreType.DMA,
            pltpu.SemaphoreType.DMA,
        ],
    )(_sc_edges)
    return f(srcr, dstr, avec, ht)


def _k3_body(opart_ref, spart_ref, avec_ref, ht_ref, w2t_ref, wb_ref,
             bat_ref, bpm_ref, x1_ref, x64_ref):
    num = opart_ref[0] + opart_ref[1]
    s_edges = jnp.sum(spart_ref[...], axis=0)
    p_self = avec_ref[2, :]
    ht = ht_ref[...]
    s_tot = s_edges + p_self + 1e-16
    xc = (num + p_self[None, :] * ht) / s_tot[None, :]
    xc = xc + wb_ref[0, 0:F][:, None]
    bpm_ref[...] = jnp.where(xc > 0, 1.0, -1.0)
    theta = jnp.minimum(bat_ref[...], 64.0) * ANG
    bpm_ref[30, :] = LAM * jnp.cos(theta)
    bpm_ref[31, :] = LAM * jnp.sin(theta)
    x1_ref[...] = jnp.sum(xc * wb_ref[1, 0:F][:, None], axis=0) + wb_ref[3, 0]
    x64 = lax.dot_general(w2t_ref[...], xc, (((1,), (0,)), ((), ())),
                          preferred_element_type=jnp.float32)
    x64_ref[...] = x64 + wb_ref[2, 0:G][:, None]


def _k4_body(bmin_ref, bmax_ref, bpm_i_ref, bpm_j_ref, dup_ref):
    i = pl.program_id(0)
    j = pl.program_id(1)

    @pl.when(j == 0)
    def _():
        dup_ref[...] = jnp.zeros((BT,), jnp.float32)

    @pl.when((j < i) & (bmax_ref[j] >= bmin_ref[i]))
    def _():
        sim = lax.dot_general(bpm_i_ref[...], bpm_j_ref[...],
                              (((0,), (0,)), ((), ())),
                              precision=lax.Precision.HIGHEST,
                              preferred_element_type=jnp.float32)
        anyj = jnp.max(jnp.where(sim > 2077.0, 1.0, 0.0), axis=1)
        dup_ref[...] = jnp.maximum(dup_ref[...], anyj)

    @pl.when(j == i)
    def _():
        sim = lax.dot_general(bpm_i_ref[...], bpm_j_ref[...],
                              (((0,), (0,)), ((), ())),
                              precision=lax.Precision.HIGHEST,
                              preferred_element_type=jnp.float32)
        li = lax.broadcasted_iota(jnp.int32, (BT, BT), 0)
        lj = lax.broadcasted_iota(jnp.int32, (BT, BT), 1)
        msk = (sim > 2077.0) & (lj < li)
        anyj = jnp.max(jnp.where(msk, 1.0, 0.0), axis=1)
        dup_ref[...] = jnp.maximum(dup_ref[...], anyj)


def _k5_body(x1_ref, x64_ref, dup_ref, bat_ref, gx_ref):
    x1 = x1_ref[...]
    pw = jnp.where(dup_ref[...] < 0.5, jnp.exp(x1), 0.0)
    bat = bat_ref[...]
    gids = lax.broadcasted_iota(jnp.int32, (G, NPAD), 0).astype(jnp.float32)
    oht = jnp.where(gids == bat[None, :], 1.0, 0.0)
    wx = x64_ref[...] * pw[None, :]
    numm = lax.dot_general(oht, wx, (((1,), (1,)), ((), ())),
                           preferred_element_type=jnp.float32)
    den = jnp.sum(oht * pw[None, :], axis=1, keepdims=True)
    gx_ref[...] = numm / (den + 1e-16)


def kernel(x, edge_index, batch, W_gat, att_src, att_dst, b_gat,
           W1, b1, W2, b2):
    src = edge_index[0]
    dst = edge_index[1]
    srcr = jnp.concatenate(
        [src, jnp.zeros((EPAD - E,), jnp.int32)]).reshape(2, EPC)
    dstr = jnp.concatenate(
        [dst, jnp.full((EPAD - E,), N, jnp.int32)]).reshape(2, EPC)
    wgt_pad = jnp.zeros((F, HID), jnp.float32).at[:30, :].set(W_gat.T)
    att2 = (jnp.zeros((2, F), jnp.float32)
            .at[0, :30].set(att_src).at[1, :30].set(att_dst))
    wb = (jnp.zeros((4, 128), jnp.float32)
          .at[0, :30].set(b_gat)
          .at[1, :30].set(W1[:, 0])
          .at[2, :G].set(b2)
          .at[3, 0].set(b1[0]))
    w2t_pad = jnp.zeros((G, F), jnp.float32).at[:, :30].set(W2.T)
    bat_f = jnp.concatenate(
        [batch.astype(jnp.float32), jnp.full((NPAD - N,), 1e9, jnp.float32)])
    bat_pad = jnp.concatenate(
        [batch, jnp.full((NPAD - N,), 2 ** 30, jnp.int32)])
    bmin = bat_pad[0::BT]
    bmax = bat_pad[BT - 1::BT]

    ht, avec = pl.pallas_call(
        _k1_body,
        out_shape=[jax.ShapeDtypeStruct((F, NPAD), jnp.float32),
                   jax.ShapeDtypeStruct((8, NPAD), jnp.float32)],
    )(x, wgt_pad, att2)

    s_part, o_part, _ = _run_sc(srcr, dstr, avec, ht)

    bpm, x1, x64 = pl.pallas_call(
        _k3_body,
        out_shape=[jax.ShapeDtypeStruct((F, NPAD), jnp.float32),
                   jax.ShapeDtypeStruct((NPAD,), jnp.float32),
                   jax.ShapeDtypeStruct((G, NPAD), jnp.float32)],
    )(o_part, s_part, avec, ht, w2t_pad, wb, bat_f)

    dup = pl.pallas_call(
        _k4_body,
        grid_spec=pltpu.PrefetchScalarGridSpec(
            num_scalar_prefetch=2,
            grid=(NB, NB),
            in_specs=[
                pl.BlockSpec((F, BT), lambda i, j, bn, bx: (0, i)),
                pl.BlockSpec((F, BT), lambda i, j, bn, bx: (0, j)),
            ],
            out_specs=pl.BlockSpec((BT,), lambda i, j, bn, bx: (i,)),
        ),
        out_shape=jax.ShapeDtypeStruct((NPAD,), jnp.float32),
    )(bmin, bmax, bpm, bpm)

    gx = pl.pallas_call(
        _k5_body,
        out_shape=jax.ShapeDtypeStruct((G, G), jnp.float32),
    )(x1, x64, dup, bat_f)
    return gx


# speedup vs baseline: 19.1365x; 1.0051x over previous
import functools
import jax
import jax.numpy as jnp
from jax import lax
from jax.experimental import pallas as pl
from jax.experimental.pallas import tpu as pltpu
from jax.experimental.pallas import tpu_sc as plsc

N = 10000
NPAD = 10240
E = 160000
HID = 256
F = 32
G = 64
NW = 32
EPT = 5120
EPAD = NW * EPT
NQ = 4
EPQ = EPAD // NQ
CE = 4096
NCH = EPQ // CE
FPT = 4
BT = 1024
NB = NPAD // BT
LAM = 45.254834
ANG = 0.09817477


def _k1_body(x_ref, wgt_ref, att_ref, ht_ref, avec_ref):
    ht = lax.dot_general(wgt_ref[...], x_ref[...],
                         (((1,), (1,)), ((), ())),
                         preferred_element_type=jnp.float32)
    ht_ref[:, 0:N] = ht
    ht_ref[:, N:NPAD] = jnp.zeros((F, NPAD - N), jnp.float32)
    a_s = jnp.sum(ht * att_ref[0, 0:F][:, None], axis=0)
    a_d = jnp.sum(ht * att_ref[1, 0:F][:, None], axis=0)
    e = a_s + a_d
    e = jnp.where(e >= 0, e, 0.2 * e)
    p_self = jnp.exp(e)
    zpad = jnp.zeros((NPAD - N,), jnp.float32)
    avec_ref[0, 0:N] = a_s
    avec_ref[0, N:NPAD] = zpad
    avec_ref[1, 0:N] = a_d
    avec_ref[1, N:NPAD] = jnp.full((NPAD - N,), -1e30, jnp.float32)
    avec_ref[2, 0:N] = p_self
    avec_ref[2, N:NPAD] = zpad


def _sc_edges(src_hbm, dst_hbm, avec_hbm, ht_hbm,
              s_out, o_out, p_out,
              as_v, ad_v, h2_v, h3_v, srcv, dstv, pv,
              s_loc, acc1, acc2, acc3,
              srcc, dstc, pc, srcc2, dstc2, pc2, sem0, sema, semb):
    c = lax.axis_index("c")
    s = lax.axis_index("s")
    wid = c * 16 + s
    wq = wid // 8
    wo = (wid % 8) * EPT
    pltpu.async_copy(avec_hbm.at[0], as_v, sem0)
    pltpu.async_copy(avec_hbm.at[1], ad_v, sem0)
    pltpu.async_copy(src_hbm.at[wq, pl.ds(wo, EPT)], srcv, sem0)
    pltpu.async_copy(dst_hbm.at[wq, pl.ds(wo, EPT)], dstv, sem0)

    zero16 = jnp.zeros((16,), jnp.float32)

    def zloop(i, _):
        for u in range(4):
            s_loc[pl.ds(i * 64 + u * 16, 16)] = zero16
        return 0
    lax.fori_loop(0, NPAD // 64, zloop, 0)

    pltpu.make_async_copy(avec_hbm.at[0], as_v, sem0).wait()
    pltpu.make_async_copy(avec_hbm.at[1], ad_v, sem0).wait()
    pltpu.make_async_copy(src_hbm.at[wq, pl.ds(wo, EPT)], srcv, sem0).wait()
    pltpu.make_async_copy(dst_hbm.at[wq, pl.ds(wo, EPT)], dstv, sem0).wait()

    def ploop(t, _):
        for u in range(2):
            off = t * 32 + u * 16
            sv = srcv[pl.ds(off, 16)]
            dv = dstv[pl.ds(off, 16)]
            a1 = plsc.load_gather(as_v, [sv])
            a2 = plsc.load_gather(ad_v, [dv])
            e = a1 + a2
            e = jnp.where(e >= 0, e, 0.2 * e)
            pe = jnp.exp(e)
            pv[pl.ds(off, 16)] = pe
            plsc.addupdate_scatter(s_loc, [dv], pe)
        return 0
    lax.fori_loop(0, EPT // 32, ploop, 0)

    pltpu.sync_copy(pv, p_out.at[wq, pl.ds(wo, EPT)])
    pltpu.sync_copy(s_loc, s_out.at[wid])
    plsc.subcore_barrier()

    f0 = (wid % 8) * FPT
    pltpu.async_copy(ht_hbm.at[f0], as_v, sem0)
    pltpu.async_copy(ht_hbm.at[f0 + 1], ad_v, sem0)
    pltpu.async_copy(ht_hbm.at[f0 + 2], h2_v, sem0)
    pltpu.async_copy(ht_hbm.at[f0 + 3], h3_v, sem0)

    def zloop2(i, _):
        for u in range(2):
            o = i * 32 + u * 16
            s_loc[pl.ds(o, 16)] = zero16
            acc1[pl.ds(o, 16)] = zero16
            acc2[pl.ds(o, 16)] = zero16
            acc3[pl.ds(o, 16)] = zero16
        return 0
    lax.fori_loop(0, NPAD // 32, zloop2, 0)

    pltpu.make_async_copy(ht_hbm.at[f0], as_v, sem0).wait()
    pltpu.make_async_copy(ht_hbm.at[f0 + 1], ad_v, sem0).wait()
    pltpu.make_async_copy(ht_hbm.at[f0 + 2], h2_v, sem0).wait()
    pltpu.make_async_copy(ht_hbm.at[f0 + 3], h3_v, sem0).wait()

    def fire(k, sb, db, pb, sem):
        pltpu.async_copy(src_hbm.at[wq, pl.ds(k * CE, CE)], sb, sem)
        pltpu.async_copy(dst_hbm.at[wq, pl.ds(k * CE, CE)], db, sem)
        pltpu.async_copy(p_out.at[wq, pl.ds(k * CE, CE)], pb, sem)

    def drain(k, sb, db, pb, sem):
        pltpu.make_async_copy(src_hbm.at[wq, pl.ds(k * CE, CE)], sb, sem).wait()
        pltpu.make_async_copy(dst_hbm.at[wq, pl.ds(k * CE, CE)], db, sem).wait()
        pltpu.make_async_copy(p_out.at[wq, pl.ds(k * CE, CE)], pb, sem).wait()

    def compute(sb, db, pb):
        def inner(i, _):
            for u in range(2):
                off = i * 32 + u * 16
                sv = sb[pl.ds(off, 16)]
                dv = db[pl.ds(off, 16)]
                pe = pb[pl.ds(off, 16)]
                g0 = plsc.load_gather(as_v, [sv])
                plsc.addupdate_scatter(s_loc, [dv], g0 * pe)
                g1 = plsc.load_gather(ad_v, [sv])
                plsc.addupdate_scatter(acc1, [dv], g1 * pe)
                g2 = plsc.load_gather(h2_v, [sv])
                plsc.addupdate_scatter(acc2, [dv], g2 * pe)
                g3 = plsc.load_gather(h3_v, [sv])
                plsc.addupdate_scatter(acc3, [dv], g3 * pe)
            return 0
        lax.fori_loop(0, CE // 32, inner, 0)

    fire(0, srcc, dstc, pc, sema)

    def chunk2(g, _):
        k0 = g * 2
        fire(k0 + 1, srcc2, dstc2, pc2, semb)
        drain(k0, srcc, dstc, pc, sema)
        compute(srcc, dstc, pc)

        @pl.when(k0 + 2 < NCH)
        def _():
            fire(k0 + 2, srcc, dstc, pc, sema)
        drain(k0 + 1, srcc2, dstc2, pc2, semb)
        compute(srcc2, dstc2, pc2)
        return 0
    lax.fori_loop(0, NCH // 2, chunk2, 0)

    pltpu.sync_copy(s_loc, o_out.at[wq, f0])
    pltpu.sync_copy(acc1, o_out.at[wq, f0 + 1])
    pltpu.sync_copy(acc2, o_out.at[wq, f0 + 2])
    pltpu.sync_copy(acc3, o_out.at[wq, f0 + 3])


def _run_sc(srcr, dstr, avec, ht):
    mesh = plsc.VectorSubcoreMesh(core_axis_name="c", subcore_axis_name="s",
                                  num_cores=2, num_subcores=16)
    f = functools.partial(
        pl.kernel,
        out_type=[jax.ShapeDtypeStruct((NW, NPAD), jnp.float32),
                  jax.ShapeDtypeStruct((NQ, F, NPAD), jnp.float32),
                  jax.ShapeDtypeStruct((NQ, EPQ), jnp.float32)],
        mesh=mesh,
        compiler_params=pltpu.CompilerParams(needs_layout_passes=False),
        scratch_types=[
            pltpu.VMEM((NPAD,), jnp.float32),
            pltpu.VMEM((NPAD,), jnp.float32),
            pltpu.VMEM((NPAD,), jnp.float32),
            pltpu.VMEM((NPAD,), jnp.float32),
            pltpu.VMEM((EPT,), jnp.int32),
            pltpu.VMEM((EPT,), jnp.int32),
            pltpu.VMEM((EPT,), jnp.float32),
            pltpu.VMEM((NPAD,), jnp.float32),
            pltpu.VMEM((NPAD,), jnp.float32),
            pltpu.VMEM((NPAD,), jnp.float32),
            pltpu.VMEM((NPAD,), jnp.float32),
            pltpu.VMEM((CE,), jnp.int32),
            pltpu.VMEM((CE,), jnp.int32),
            pltpu.VMEM((CE,), jnp.float32),
            pltpu.VMEM((CE,), jnp.int32),
            pltpu.VMEM((CE,), jnp.int32),
            pltpu.VMEM((CE,), jnp.float32),
            pltpu.SemaphoreType.DMA,
            pltpu.SemaphoreType.DMA,
            pltpu.SemaphoreType.DMA,
        ],
    )(_sc_edges)
    return f(srcr, dstr, avec, ht)


def _k3_body(opart_ref, spart_ref, avec_ref, ht_ref, w2t_ref, wb_ref,
             bat_ref, bpm_ref, x1_ref, x64_ref):
    num = (opart_ref[0] + opart_ref[1]) + (opart_ref[2] + opart_ref[3])
    s_edges = jnp.sum(spart_ref[...], axis=0)
    p_self = avec_ref[2, :]
    ht = ht_ref[...]
    s_tot = s_edges + p_self + 1e-16
    xc = (num + p_self[None, :] * ht) / s_tot[None, :]
    xc = xc + wb_ref[0, 0:F][:, None]
    bpm_ref[...] = jnp.where(xc > 0, 1.0, -1.0)
    theta = jnp.minimum(bat_ref[...], 64.0) * ANG
    bpm_ref[30, :] = LAM * jnp.cos(theta)
    bpm_ref[31, :] = LAM * jnp.sin(theta)
    x1_ref[...] = jnp.sum(xc * wb_ref[1, 0:F][:, None], axis=0) + wb_ref[3, 0]
    x64 = lax.dot_general(w2t_ref[...], xc, (((1,), (0,)), ((), ())),
                          preferred_element_type=jnp.float32)
    x64_ref[...] = x64 + wb_ref[2, 0:G][:, None]


def _k4_body(bmin_ref, bmax_ref, bpm_i_ref, bpm_j_ref, dup_ref, dmask_ref):
    i = pl.program_id(0)
    j = pl.program_id(1)

    @pl.when(j == 0)
    def _():
        dup_ref[...] = jnp.zeros((BT,), jnp.float32)

    @pl.when((i == 0) & (j == 0))
    def _():
        li = lax.broadcasted_iota(jnp.int32, (BT, BT), 0)
        lj = lax.broadcasted_iota(jnp.int32, (BT, BT), 1)
        dmask_ref[...] = jnp.where(lj < li, 0.0, -8192.0)

    @pl.when((j < i) & (bmax_ref[j] >= bmin_ref[i]))
    def _():
        sim = lax.dot_general(bpm_i_ref[...], bpm_j_ref[...],
                              (((0,), (0,)), ((), ())),
                              precision=lax.Precision.HIGHEST,
                              preferred_element_type=jnp.float32)
        rm = jnp.max(sim, axis=1)
        dup_ref[...] = jnp.maximum(dup_ref[...],
                                   jnp.where(rm > 2077.0, 1.0, 0.0))

    @pl.when(j == i)
    def _():
        sim = lax.dot_general(bpm_i_ref[...], bpm_j_ref[...],
                              (((0,), (0,)), ((), ())),
                              precision=lax.Precision.HIGHEST,
                              preferred_element_type=jnp.float32)
        rm = jnp.max(sim + dmask_ref[...], axis=1)
        dup_ref[...] = jnp.maximum(dup_ref[...],
                                   jnp.where(rm > 2077.0, 1.0, 0.0))


def _k5_body(x1_ref, x64_ref, dup_ref, bat_ref, gx_ref):
    x1 = x1_ref[...]
    pw = jnp.where(dup_ref[...] < 0.5, jnp.exp(x1), 0.0)
    bat = bat_ref[...]
    gids = lax.broadcasted_iota(jnp.int32, (G, NPAD), 0).astype(jnp.float32)
    oht = jnp.where(gids == bat[None, :], 1.0, 0.0)
    wx = x64_ref[...] * pw[None, :]
    numm = lax.dot_general(oht, wx, (((1,), (1,)), ((), ())),
                           preferred_element_type=jnp.float32)
    den = jnp.sum(oht * pw[None, :], axis=1, keepdims=True)
    gx_ref[...] = numm / (den + 1e-16)


def kernel(x, edge_index, batch, W_gat, att_src, att_dst, b_gat,
           W1, b1, W2, b2):
    src = edge_index[0]
    dst = edge_index[1]
    srcr = jnp.concatenate(
        [src, jnp.zeros((EPAD - E,), jnp.int32)]).reshape(NQ, EPQ)
    dstr = jnp.concatenate(
        [dst, jnp.full((EPAD - E,), N, jnp.int32)]).reshape(NQ, EPQ)
    wgt_pad = jnp.zeros((F, HID), jnp.float32).at[:30, :].set(W_gat.T)
    att2 = (jnp.zeros((2, F), jnp.float32)
            .at[0, :30].set(att_src).at[1, :30].set(att_dst))
    wb = (jnp.zeros((4, 128), jnp.float32)
          .at[0, :30].set(b_gat)
          .at[1, :30].set(W1[:, 0])
          .at[2, :G].set(b2)
          .at[3, 0].set(b1[0]))
    w2t_pad = jnp.zeros((G, F), jnp.float32).at[:, :30].set(W2.T)
    bat_f = jnp.concatenate(
        [batch.astype(jnp.float32), jnp.full((NPAD - N,), 1e9, jnp.float32)])
    bat_pad = jnp.concatenate(
        [batch, jnp.full((NPAD - N,), 2 ** 30, jnp.int32)])
    bmin = bat_pad[0::BT]
    bmax = bat_pad[BT - 1::BT]

    ht, avec = pl.pallas_call(
        _k1_body,
        out_shape=[jax.ShapeDtypeStruct((F, NPAD), jnp.float32),
                   jax.ShapeDtypeStruct((8, NPAD), jnp.float32)],
    )(x, wgt_pad, att2)

    s_part, o_part, _ = _run_sc(srcr, dstr, avec, ht)

    bpm, x1, x64 = pl.pallas_call(
        _k3_body,
        out_shape=[jax.ShapeDtypeStruct((F, NPAD), jnp.float32),
                   jax.ShapeDtypeStruct((NPAD,), jnp.float32),
                   jax.ShapeDtypeStruct((G, NPAD), jnp.float32)],
    )(o_part, s_part, avec, ht, w2t_pad, wb, bat_f)

    dup = pl.pallas_call(
        _k4_body,
        grid_spec=pltpu.PrefetchScalarGridSpec(
            num_scalar_prefetch=2,
            grid=(NB, NB),
            in_specs=[
                pl.BlockSpec((F, BT), lambda i, j, bn, bx: (0, i)),
                pl.BlockSpec((F, BT), lambda i, j, bn, bx: (0, j)),
            ],
            out_specs=pl.BlockSpec((BT,), lambda i, j, bn, bx: (i,)),
            scratch_shapes=[pltpu.VMEM((BT, BT), jnp.float32)],
        ),
        out_shape=jax.ShapeDtypeStruct((NPAD,), jnp.float32),
    )(bmin, bmax, bpm, bpm)

    gx = pl.pallas_call(
        _k5_body,
        out_shape=jax.ShapeDtypeStruct((G, G), jnp.float32),
    )(x1, x64, dup, bat_f)
    return gx


# speedup vs baseline: 23.3957x; 1.2226x over previous
import functools
import jax
import jax.numpy as jnp
from jax import lax
from jax.experimental import pallas as pl
from jax.experimental.pallas import tpu as pltpu
from jax.experimental.pallas import tpu_sc as plsc

N = 10000
NPAD = 10240
E = 160000
HID = 256
F = 32
G = 64
NW = 32
EPT = 5120
EPAD = NW * EPT
NQ = 4
EPQ = EPAD // NQ
CE = 4096
NCH = EPQ // CE
FPT = 4
BT = 1024
NB = NPAD // BT
F2 = 48


def _k1_body(x_ref, wgt_ref, att_ref, ht_ref, avec_ref):
    ht = lax.dot_general(wgt_ref[...], x_ref[...],
                         (((1,), (1,)), ((), ())),
                         preferred_element_type=jnp.float32)
    ht_ref[:, 0:N] = ht
    ht_ref[:, N:NPAD] = jnp.zeros((F, NPAD - N), jnp.float32)
    a_s = jnp.sum(ht * att_ref[0, 0:F][:, None], axis=0)
    a_d = jnp.sum(ht * att_ref[1, 0:F][:, None], axis=0)
    e = a_s + a_d
    e = jnp.where(e >= 0, e, 0.2 * e)
    p_self = jnp.exp(e)
    zpad = jnp.zeros((NPAD - N,), jnp.float32)
    avec_ref[0, 0:N] = a_s
    avec_ref[0, N:NPAD] = zpad
    avec_ref[1, 0:N] = a_d
    avec_ref[1, N:NPAD] = jnp.full((NPAD - N,), -1e30, jnp.float32)
    avec_ref[2, 0:N] = p_self
    avec_ref[2, N:NPAD] = zpad


def _sc_edges(src_hbm, dst_hbm, avec_hbm, ht_hbm,
              s_out, o_out, p_out,
              as_v, ad_v, h2_v, h3_v, srcv, dstv, pv,
              s_loc, acc1, acc2, acc3,
              srcc, dstc, pc, srcc2, dstc2, pc2, sem0, sema, semb):
    c = lax.axis_index("c")
    s = lax.axis_index("s")
    wid = c * 16 + s
    wq = wid // 8
    wo = (wid % 8) * EPT
    pltpu.async_copy(avec_hbm.at[0], as_v, sem0)
    pltpu.async_copy(avec_hbm.at[1], ad_v, sem0)
    pltpu.async_copy(src_hbm.at[wq, pl.ds(wo, EPT)], srcv, sem0)
    pltpu.async_copy(dst_hbm.at[wq, pl.ds(wo, EPT)], dstv, sem0)

    zero16 = jnp.zeros((16,), jnp.float32)

    def zloop(i, _):
        for u in range(4):
            s_loc[pl.ds(i * 64 + u * 16, 16)] = zero16
        return 0
    lax.fori_loop(0, NPAD // 64, zloop, 0)

    pltpu.make_async_copy(avec_hbm.at[0], as_v, sem0).wait()
    pltpu.make_async_copy(avec_hbm.at[1], ad_v, sem0).wait()
    pltpu.make_async_copy(src_hbm.at[wq, pl.ds(wo, EPT)], srcv, sem0).wait()
    pltpu.make_async_copy(dst_hbm.at[wq, pl.ds(wo, EPT)], dstv, sem0).wait()

    def ploop(t, _):
        for u in range(2):
            off = t * 32 + u * 16
            sv = srcv[pl.ds(off, 16)]
            dv = dstv[pl.ds(off, 16)]
            a1 = plsc.load_gather(as_v, [sv])
            a2 = plsc.load_gather(ad_v, [dv])
            e = a1 + a2
            e = jnp.where(e >= 0, e, 0.2 * e)
            pe = jnp.exp(e)
            pv[pl.ds(off, 16)] = pe
            plsc.addupdate_scatter(s_loc, [dv], pe)
        return 0
    lax.fori_loop(0, EPT // 32, ploop, 0)

    pltpu.sync_copy(pv, p_out.at[wq, pl.ds(wo, EPT)])
    pltpu.sync_copy(s_loc, s_out.at[wid])
    plsc.subcore_barrier()

    f0 = (wid % 8) * FPT
    pltpu.async_copy(ht_hbm.at[f0], as_v, sem0)
    pltpu.async_copy(ht_hbm.at[f0 + 1], ad_v, sem0)
    pltpu.async_copy(ht_hbm.at[f0 + 2], h2_v, sem0)
    pltpu.async_copy(ht_hbm.at[f0 + 3], h3_v, sem0)

    def zloop2(i, _):
        for u in range(2):
            o = i * 32 + u * 16
            s_loc[pl.ds(o, 16)] = zero16
            acc1[pl.ds(o, 16)] = zero16
            acc2[pl.ds(o, 16)] = zero16
            acc3[pl.ds(o, 16)] = zero16
        return 0
    lax.fori_loop(0, NPAD // 32, zloop2, 0)

    pltpu.make_async_copy(ht_hbm.at[f0], as_v, sem0).wait()
    pltpu.make_async_copy(ht_hbm.at[f0 + 1], ad_v, sem0).wait()
    pltpu.make_async_copy(ht_hbm.at[f0 + 2], h2_v, sem0).wait()
    pltpu.make_async_copy(ht_hbm.at[f0 + 3], h3_v, sem0).wait()

    def fire(k, sb, db, pb, sem):
        pltpu.async_copy(src_hbm.at[wq, pl.ds(k * CE, CE)], sb, sem)
        pltpu.async_copy(dst_hbm.at[wq, pl.ds(k * CE, CE)], db, sem)
        pltpu.async_copy(p_out.at[wq, pl.ds(k * CE, CE)], pb, sem)

    def drain(k, sb, db, pb, sem):
        pltpu.make_async_copy(src_hbm.at[wq, pl.ds(k * CE, CE)], sb, sem).wait()
        pltpu.make_async_copy(dst_hbm.at[wq, pl.ds(k * CE, CE)], db, sem).wait()
        pltpu.make_async_copy(p_out.at[wq, pl.ds(k * CE, CE)], pb, sem).wait()

    def compute(sb, db, pb):
        def inner(i, _):
            for u in range(2):
                off = i * 32 + u * 16
                sv = sb[pl.ds(off, 16)]
                dv = db[pl.ds(off, 16)]
                pe = pb[pl.ds(off, 16)]
                g0 = plsc.load_gather(as_v, [sv])
                plsc.addupdate_scatter(s_loc, [dv], g0 * pe)
                g1 = plsc.load_gather(ad_v, [sv])
                plsc.addupdate_scatter(acc1, [dv], g1 * pe)
                g2 = plsc.load_gather(h2_v, [sv])
                plsc.addupdate_scatter(acc2, [dv], g2 * pe)
                g3 = plsc.load_gather(h3_v, [sv])
                plsc.addupdate_scatter(acc3, [dv], g3 * pe)
            return 0
        lax.fori_loop(0, CE // 32, inner, 0)

    fire(0, srcc, dstc, pc, sema)

    def chunk2(g, _):
        k0 = g * 2
        fire(k0 + 1, srcc2, dstc2, pc2, semb)
        drain(k0, srcc, dstc, pc, sema)
        compute(srcc, dstc, pc)

        @pl.when(k0 + 2 < NCH)
        def _():
            fire(k0 + 2, srcc, dstc, pc, sema)
        drain(k0 + 1, srcc2, dstc2, pc2, semb)
        compute(srcc2, dstc2, pc2)
        return 0
    lax.fori_loop(0, NCH // 2, chunk2, 0)

    pltpu.sync_copy(s_loc, o_out.at[wq, f0])
    pltpu.sync_copy(acc1, o_out.at[wq, f0 + 1])
    pltpu.sync_copy(acc2, o_out.at[wq, f0 + 2])
    pltpu.sync_copy(acc3, o_out.at[wq, f0 + 3])


def _run_sc(srcr, dstr, avec, ht):
    mesh = plsc.VectorSubcoreMesh(core_axis_name="c", subcore_axis_name="s",
                                  num_cores=2, num_subcores=16)
    f = functools.partial(
        pl.kernel,
        out_type=[jax.ShapeDtypeStruct((NW, NPAD), jnp.float32),
                  jax.ShapeDtypeStruct((NQ, F, NPAD), jnp.float32),
                  jax.ShapeDtypeStruct((NQ, EPQ), jnp.float32)],
        mesh=mesh,
        compiler_params=pltpu.CompilerParams(needs_layout_passes=False),
        scratch_types=[
            pltpu.VMEM((NPAD,), jnp.float32),
            pltpu.VMEM((NPAD,), jnp.float32),
            pltpu.VMEM((NPAD,), jnp.float32),
            pltpu.VMEM((NPAD,), jnp.float32),
            pltpu.VMEM((EPT,), jnp.int32),
            pltpu.VMEM((EPT,), jnp.int32),
            pltpu.VMEM((EPT,), jnp.float32),
            pltpu.VMEM((NPAD,), jnp.float32),
            pltpu.VMEM((NPAD,), jnp.float32),
            pltpu.VMEM((NPAD,), jnp.float32),
            pltpu.VMEM((NPAD,), jnp.float32),
            pltpu.VMEM((CE,), jnp.int32),
            pltpu.VMEM((CE,), jnp.int32),
            pltpu.VMEM((CE,), jnp.float32),
            pltpu.VMEM((CE,), jnp.int32),
            pltpu.VMEM((CE,), jnp.int32),
            pltpu.VMEM((CE,), jnp.float32),
            pltpu.SemaphoreType.DMA,
            pltpu.SemaphoreType.DMA,
            pltpu.SemaphoreType.DMA,
        ],
    )(_sc_edges)
    return f(srcr, dstr, avec, ht)


def _k3_body(opart_ref, spart_ref, avec_ref, ht_ref, w2t_ref, wb_ref,
             bat_ref, bpm_ref, x1_ref, x64_ref):
    num = (opart_ref[0] + opart_ref[1]) + (opart_ref[2] + opart_ref[3])
    s_edges = jnp.sum(spart_ref[...], axis=0)
    p_self = avec_ref[2, :]
    ht = ht_ref[...]
    s_tot = s_edges + p_self + 1e-16
    xc = (num + p_self[None, :] * ht) / s_tot[None, :]
    xc = xc + wb_ref[0, 0:F][:, None]
    sg = jnp.where(xc > 0, 1.0, -1.0)
    bc = jnp.minimum(bat_ref[...], 63.0)
    brows = []
    for k in range(6):
        q = jnp.floor(bc * 0.5)
        brows.append((2.0 * (bc - 2.0 * q) - 1.0)[None, :])
        bc = q
    zrows = jnp.zeros((F2 - F - 6, NPAD), jnp.float32)
    full = jnp.concatenate([sg] + brows + [zrows], axis=0)
    bpm_ref[...] = full.astype(jnp.bfloat16)
    x1_ref[...] = jnp.sum(xc * wb_ref[1, 0:F][:, None], axis=0) + wb_ref[3, 0]
    x64 = lax.dot_general(w2t_ref[...], xc, (((1,), (0,)), ((), ())),
                          preferred_element_type=jnp.float32)
    x64_ref[...] = x64 + wb_ref[2, 0:G][:, None]


def _k4_body(bmin_ref, bmax_ref, bpm_i_ref, bpm_j_ref, dup_ref, dmask_ref):
    i = pl.program_id(0)
    j = pl.program_id(1)

    @pl.when(j == 0)
    def _():
        dup_ref[...] = jnp.zeros((BT,), jnp.float32)

    @pl.when((i == 0) & (j == 0))
    def _():
        li = lax.broadcasted_iota(jnp.int32, (BT, BT), 0)
        lj = lax.broadcasted_iota(jnp.int32, (BT, BT), 1)
        dmask_ref[...] = jnp.where(lj < li, 0.0, -8192.0)

    @pl.when((j < i) & (bmax_ref[j] >= bmin_ref[i]))
    def _():
        sim = lax.dot_general(bpm_i_ref[...], bpm_j_ref[...],
                              (((0,), (0,)), ((), ())),
                              preferred_element_type=jnp.float32)
        rm = jnp.max(sim, axis=1)
        dup_ref[...] = jnp.maximum(dup_ref[...],
                                   jnp.where(rm > 37.0, 1.0, 0.0))

    @pl.when(j == i)
    def _():
        sim = lax.dot_general(bpm_i_ref[...], bpm_j_ref[...],
                              (((0,), (0,)), ((), ())),
                              preferred_element_type=jnp.float32)
        rm = jnp.max(sim + dmask_ref[...], axis=1)
        dup_ref[...] = jnp.maximum(dup_ref[...],
                                   jnp.where(rm > 37.0, 1.0, 0.0))


def _k5_body(x1_ref, x64_ref, dup_ref, bat_ref, gx_ref):
    x1 = x1_ref[...]
    pw = jnp.where(dup_ref[...] < 0.5, jnp.exp(x1), 0.0)
    bat = bat_ref[...]
    gids = lax.broadcasted_iota(jnp.int32, (G, NPAD), 0).astype(jnp.float32)
    oht = jnp.where(gids == bat[None, :], 1.0, 0.0)
    wx = x64_ref[...] * pw[None, :]
    numm = lax.dot_general(oht, wx, (((1,), (1,)), ((), ())),
                           preferred_element_type=jnp.float32)
    den = jnp.sum(oht * pw[None, :], axis=1, keepdims=True)
    gx_ref[...] = numm / (den + 1e-16)


def kernel(x, edge_index, batch, W_gat, att_src, att_dst, b_gat,
           W1, b1, W2, b2):
    src = edge_index[0]
    dst = edge_index[1]
    srcr = jnp.concatenate(
        [src, jnp.zeros((EPAD - E,), jnp.int32)]).reshape(NQ, EPQ)
    dstr = jnp.concatenate(
        [dst, jnp.full((EPAD - E,), N, jnp.int32)]).reshape(NQ, EPQ)
    wgt_pad = jnp.zeros((F, HID), jnp.float32).at[:30, :].set(W_gat.T)
    att2 = (jnp.zeros((2, F), jnp.float32)
            .at[0, :30].set(att_src).at[1, :30].set(att_dst))
    wb = (jnp.zeros((4, 128), jnp.float32)
          .at[0, :30].set(b_gat)
          .at[1, :30].set(W1[:, 0])
          .at[2, :G].set(b2)
          .at[3, 0].set(b1[0]))
    w2t_pad = jnp.zeros((G, F), jnp.float32).at[:, :30].set(W2.T)
    bat_f = jnp.concatenate(
        [batch.astype(jnp.float32), jnp.full((NPAD - N,), 1e9, jnp.float32)])
    bat_pad = jnp.concatenate(
        [batch, jnp.full((NPAD - N,), 2 ** 30, jnp.int32)])
    bmin = bat_pad[0::BT]
    bmax = bat_pad[BT - 1::BT]

    ht, avec = pl.pallas_call(
        _k1_body,
        out_shape=[jax.ShapeDtypeStruct((F, NPAD), jnp.float32),
                   jax.ShapeDtypeStruct((8, NPAD), jnp.float32)],
    )(x, wgt_pad, att2)

    s_part, o_part, _ = _run_sc(srcr, dstr, avec, ht)

    bpm, x1, x64 = pl.pallas_call(
        _k3_body,
        out_shape=[jax.ShapeDtypeStruct((F2, NPAD), jnp.bfloat16),
                   jax.ShapeDtypeStruct((NPAD,), jnp.float32),
                   jax.ShapeDtypeStruct((G, NPAD), jnp.float32)],
    )(o_part, s_part, avec, ht, w2t_pad, wb, bat_f)

    dup = pl.pallas_call(
        _k4_body,
        grid_spec=pltpu.PrefetchScalarGridSpec(
            num_scalar_prefetch=2,
            grid=(NB, NB),
            in_specs=[
                pl.BlockSpec((F2, BT), lambda i, j, bn, bx: (0, i)),
                pl.BlockSpec((F2, BT), lambda i, j, bn, bx: (0, j)),
            ],
            out_specs=pl.BlockSpec((BT,), lambda i, j, bn, bx: (i,)),
            scratch_shapes=[pltpu.VMEM((BT, BT), jnp.float32)],
        ),
        out_shape=jax.ShapeDtypeStruct((NPAD,), jnp.float32),
    )(bmin, bmax, bpm, bpm)

    gx = pl.pallas_call(
        _k5_body,
        out_shape=jax.ShapeDtypeStruct((G, G), jnp.float32),
    )(x1, x64, dup, bat_f)
    return gx


# speedup vs baseline: 31.6529x; 1.3529x over previous
import functools
import jax
import jax.numpy as jnp
from jax import lax
from jax.experimental import pallas as pl
from jax.experimental.pallas import tpu as pltpu
from jax.experimental.pallas import tpu_sc as plsc

N = 10000
NPAD = 10240
E = 160000
HID = 256
F = 32
G = 64
NW = 32
EPT = 5120
EPAD = NW * EPT
NQ = 4
EPQ = EPAD // NQ
CE = 4096
NCH = EPQ // CE
FPT = 4
BT = 1024
NB = NPAD // BT
F2 = 48


def _k1_body(x_ref, wgt_ref, att_ref, ei_ref, ht_ref, avec_ref,
             srcr_ref, dstr_ref):
    ER = E - 3 * EPQ
    for q in range(NQ - 1):
        srcr_ref[q, :] = ei_ref[0, pl.ds(q * EPQ, EPQ)]
        dstr_ref[q, :] = ei_ref[1, pl.ds(q * EPQ, EPQ)]
    srcr_ref[3, 0:ER] = ei_ref[0, pl.ds(3 * EPQ, ER)]
    srcr_ref[3, ER:EPQ] = jnp.zeros((EPQ - ER,), jnp.int32)
    dstr_ref[3, 0:ER] = ei_ref[1, pl.ds(3 * EPQ, ER)]
    dstr_ref[3, ER:EPQ] = jnp.full((EPQ - ER,), N, jnp.int32)
    ht = lax.dot_general(wgt_ref[...], x_ref[...],
                         (((1,), (1,)), ((), ())),
                         preferred_element_type=jnp.float32)
    ht_ref[:, 0:N] = ht
    ht_ref[:, N:NPAD] = jnp.zeros((F, NPAD - N), jnp.float32)
    a_s = jnp.sum(ht * att_ref[0, 0:F][:, None], axis=0)
    a_d = jnp.sum(ht * att_ref[1, 0:F][:, None], axis=0)
    e = a_s + a_d
    e = jnp.where(e >= 0, e, 0.2 * e)
    p_self = jnp.exp(e)
    zpad = jnp.zeros((NPAD - N,), jnp.float32)
    avec_ref[0, 0:N] = a_s
    avec_ref[0, N:NPAD] = zpad
    avec_ref[1, 0:N] = a_d
    avec_ref[1, N:NPAD] = jnp.full((NPAD - N,), -1e30, jnp.float32)
    avec_ref[2, 0:N] = p_self
    avec_ref[2, N:NPAD] = zpad


def _sc_edges(src_hbm, dst_hbm, avec_hbm, ht_hbm,
              s_out, o_out, p_out,
              as_v, ad_v, h2_v, h3_v, srcv, dstv, pv,
              s_loc, acc1, acc2, acc3,
              srcc, dstc, pc, srcc2, dstc2, pc2, sem0, sema, semb):
    c = lax.axis_index("c")
    s = lax.axis_index("s")
    wid = c * 16 + s
    wq = wid // 8
    wo = (wid % 8) * EPT
    pltpu.async_copy(avec_hbm.at[0], as_v, sem0)
    pltpu.async_copy(avec_hbm.at[1], ad_v, sem0)
    pltpu.async_copy(src_hbm.at[wq, pl.ds(wo, EPT)], srcv, sem0)
    pltpu.async_copy(dst_hbm.at[wq, pl.ds(wo, EPT)], dstv, sem0)

    zero16 = jnp.zeros((16,), jnp.float32)

    def zloop(i, _):
        for u in range(4):
            s_loc[pl.ds(i * 64 + u * 16, 16)] = zero16
        return 0
    lax.fori_loop(0, NPAD // 64, zloop, 0)

    pltpu.make_async_copy(avec_hbm.at[0], as_v, sem0).wait()
    pltpu.make_async_copy(avec_hbm.at[1], ad_v, sem0).wait()
    pltpu.make_async_copy(src_hbm.at[wq, pl.ds(wo, EPT)], srcv, sem0).wait()
    pltpu.make_async_copy(dst_hbm.at[wq, pl.ds(wo, EPT)], dstv, sem0).wait()

    @plsc.parallel_loop(0, EPT // 16, 1, unroll=8)
    def ploop(t):
        off = t * 16
        sv = srcv[pl.ds(off, 16)]
        dv = dstv[pl.ds(off, 16)]
        a1 = plsc.load_gather(as_v, [sv])
        a2 = plsc.load_gather(ad_v, [dv])
        e = a1 + a2
        e = jnp.where(e >= 0, e, 0.2 * e)
        pe = jnp.exp(e)
        pv[pl.ds(off, 16)] = pe
        plsc.addupdate_scatter(s_loc, [dv], pe)

    pltpu.sync_copy(pv, p_out.at[wq, pl.ds(wo, EPT)])
    pltpu.sync_copy(s_loc, s_out.at[wid])
    plsc.subcore_barrier()

    f0 = (wid % 8) * FPT
    pltpu.async_copy(ht_hbm.at[f0], as_v, sem0)
    pltpu.async_copy(ht_hbm.at[f0 + 1], ad_v, sem0)
    pltpu.async_copy(ht_hbm.at[f0 + 2], h2_v, sem0)
    pltpu.async_copy(ht_hbm.at[f0 + 3], h3_v, sem0)

    def zloop2(i, _):
        for u in range(2):
            o = i * 32 + u * 16
            s_loc[pl.ds(o, 16)] = zero16
            acc1[pl.ds(o, 16)] = zero16
            acc2[pl.ds(o, 16)] = zero16
            acc3[pl.ds(o, 16)] = zero16
        return 0
    lax.fori_loop(0, NPAD // 32, zloop2, 0)

    pltpu.make_async_copy(ht_hbm.at[f0], as_v, sem0).wait()
    pltpu.make_async_copy(ht_hbm.at[f0 + 1], ad_v, sem0).wait()
    pltpu.make_async_copy(ht_hbm.at[f0 + 2], h2_v, sem0).wait()
    pltpu.make_async_copy(ht_hbm.at[f0 + 3], h3_v, sem0).wait()

    def fire(k, sb, db, pb, sem):
        pltpu.async_copy(src_hbm.at[wq, pl.ds(k * CE, CE)], sb, sem)
        pltpu.async_copy(dst_hbm.at[wq, pl.ds(k * CE, CE)], db, sem)
        pltpu.async_copy(p_out.at[wq, pl.ds(k * CE, CE)], pb, sem)

    def drain(k, sb, db, pb, sem):
        pltpu.make_async_copy(src_hbm.at[wq, pl.ds(k * CE, CE)], sb, sem).wait()
        pltpu.make_async_copy(dst_hbm.at[wq, pl.ds(k * CE, CE)], db, sem).wait()
        pltpu.make_async_copy(p_out.at[wq, pl.ds(k * CE, CE)], pb, sem).wait()

    def compute(sb, db, pb):
        @plsc.parallel_loop(0, CE // 16, 1, unroll=8)
        def inner(i):
            off = i * 16
            sv = sb[pl.ds(off, 16)]
            dv = db[pl.ds(off, 16)]
            pe = pb[pl.ds(off, 16)]
            g0 = plsc.load_gather(as_v, [sv])
            plsc.addupdate_scatter(s_loc, [dv], g0 * pe)
            g1 = plsc.load_gather(ad_v, [sv])
            plsc.addupdate_scatter(acc1, [dv], g1 * pe)
            g2 = plsc.load_gather(h2_v, [sv])
            plsc.addupdate_scatter(acc2, [dv], g2 * pe)
            g3 = plsc.load_gather(h3_v, [sv])
            plsc.addupdate_scatter(acc3, [dv], g3 * pe)

    fire(0, srcc, dstc, pc, sema)

    def chunk2(g, _):
        k0 = g * 2
        fire(k0 + 1, srcc2, dstc2, pc2, semb)
        drain(k0, srcc, dstc, pc, sema)
        compute(srcc, dstc, pc)

        @pl.when(k0 + 2 < NCH)
        def _():
            fire(k0 + 2, srcc, dstc, pc, sema)
        drain(k0 + 1, srcc2, dstc2, pc2, semb)
        compute(srcc2, dstc2, pc2)
        return 0
    lax.fori_loop(0, NCH // 2, chunk2, 0)

    pltpu.sync_copy(s_loc, o_out.at[wq, f0])
    pltpu.sync_copy(acc1, o_out.at[wq, f0 + 1])
    pltpu.sync_copy(acc2, o_out.at[wq, f0 + 2])
    pltpu.sync_copy(acc3, o_out.at[wq, f0 + 3])


def _run_sc(srcr, dstr, avec, ht):
    mesh = plsc.VectorSubcoreMesh(core_axis_name="c", subcore_axis_name="s",
                                  num_cores=2, num_subcores=16)
    f = functools.partial(
        pl.kernel,
        out_type=[jax.ShapeDtypeStruct((NW, NPAD), jnp.float32),
                  jax.ShapeDtypeStruct((NQ, F, NPAD), jnp.float32),
                  jax.ShapeDtypeStruct((NQ, EPQ), jnp.float32)],
        mesh=mesh,
        compiler_params=pltpu.CompilerParams(needs_layout_passes=False),
        scratch_types=[
            pltpu.VMEM((NPAD,), jnp.float32),
            pltpu.VMEM((NPAD,), jnp.float32),
            pltpu.VMEM((NPAD,), jnp.float32),
            pltpu.VMEM((NPAD,), jnp.float32),
            pltpu.VMEM((EPT,), jnp.int32),
            pltpu.VMEM((EPT,), jnp.int32),
            pltpu.VMEM((EPT,), jnp.float32),
            pltpu.VMEM((NPAD,), jnp.float32),
            pltpu.VMEM((NPAD,), jnp.float32),
            pltpu.VMEM((NPAD,), jnp.float32),
            pltpu.VMEM((NPAD,), jnp.float32),
            pltpu.VMEM((CE,), jnp.int32),
            pltpu.VMEM((CE,), jnp.int32),
            pltpu.VMEM((CE,), jnp.float32),
            pltpu.VMEM((CE,), jnp.int32),
            pltpu.VMEM((CE,), jnp.int32),
            pltpu.VMEM((CE,), jnp.float32),
            pltpu.SemaphoreType.DMA,
            pltpu.SemaphoreType.DMA,
            pltpu.SemaphoreType.DMA,
        ],
    )(_sc_edges)
    return f(srcr, dstr, avec, ht)


def _k3_body(opart_ref, spart_ref, avec_ref, ht_ref, w2t_ref, wb_ref,
             bat_ref, bpm_ref, x1_ref, x64_ref):
    num = (opart_ref[0] + opart_ref[1]) + (opart_ref[2] + opart_ref[3])
    s_edges = jnp.sum(spart_ref[...], axis=0)
    p_self = avec_ref[2, :]
    ht = ht_ref[...]
    s_tot = s_edges + p_self + 1e-16
    xc = (num + p_self[None, :] * ht) / s_tot[None, :]
    xc = xc + wb_ref[0, 0:F][:, None]
    sg = jnp.where(xc > 0, 1.0, -1.0)
    bc = jnp.minimum(bat_ref[...], 63.0)
    brows = []
    for k in range(6):
        q = jnp.floor(bc * 0.5)
        brows.append((2.0 * (bc - 2.0 * q) - 1.0)[None, :])
        bc = q
    zrows = jnp.zeros((F2 - F - 6, NPAD), jnp.float32)
    full = jnp.concatenate([sg] + brows + [zrows], axis=0)
    bpm_ref[...] = full.astype(jnp.bfloat16)
    x1_ref[...] = jnp.sum(xc * wb_ref[1, 0:F][:, None], axis=0) + wb_ref[3, 0]
    x64 = lax.dot_general(w2t_ref[...], xc, (((1,), (0,)), ((), ())),
                          preferred_element_type=jnp.float32)
    x64_ref[...] = x64 + wb_ref[2, 0:G][:, None]


def _k4_body(bmin_ref, bmax_ref, bpm_i_ref, bpm_j_ref, dup_ref, dmask_ref):
    i = pl.program_id(0)
    j = pl.program_id(1)

    @pl.when(j == 0)
    def _():
        dup_ref[...] = jnp.zeros((BT,), jnp.float32)

    @pl.when((i == 0) & (j == 0))
    def _():
        li = lax.broadcasted_iota(jnp.int32, (BT, BT), 0)
        lj = lax.broadcasted_iota(jnp.int32, (BT, BT), 1)
        dmask_ref[...] = jnp.where(lj < li, 0.0, -8192.0)

    @pl.when((j < i) & (bmax_ref[j] >= bmin_ref[i]))
    def _():
        sim = lax.dot_general(bpm_i_ref[...], bpm_j_ref[...],
                              (((0,), (0,)), ((), ())),
                              preferred_element_type=jnp.float32)
        rm = jnp.max(sim, axis=1)
        dup_ref[...] = jnp.maximum(dup_ref[...],
                                   jnp.where(rm > 37.0, 1.0, 0.0))

    @pl.when(j == i)
    def _():
        sim = lax.dot_general(bpm_i_ref[...], bpm_j_ref[...],
                              (((0,), (0,)), ((), ())),
                              preferred_element_type=jnp.float32)
        rm = jnp.max(sim + dmask_ref[...], axis=1)
        dup_ref[...] = jnp.maximum(dup_ref[...],
                                   jnp.where(rm > 37.0, 1.0, 0.0))


def _k5_body(x1_ref, x64_ref, dup_ref, bat_ref, gx_ref):
    x1 = x1_ref[...]
    pw = jnp.where(dup_ref[...] < 0.5, jnp.exp(x1), 0.0)
    bat = bat_ref[...]
    gids = lax.broadcasted_iota(jnp.int32, (G, NPAD), 0).astype(jnp.float32)
    oht = jnp.where(gids == bat[None, :], 1.0, 0.0)
    wx = x64_ref[...] * pw[None, :]
    numm = lax.dot_general(oht, wx, (((1,), (1,)), ((), ())),
                           preferred_element_type=jnp.float32)
    den = jnp.sum(oht * pw[None, :], axis=1, keepdims=True)
    gx_ref[...] = numm / (den + 1e-16)


def kernel(x, edge_index, batch, W_gat, att_src, att_dst, b_gat,
           W1, b1, W2, b2):
    wgt_pad = jnp.zeros((F, HID), jnp.float32).at[:30, :].set(W_gat.T)
    att2 = (jnp.zeros((2, F), jnp.float32)
            .at[0, :30].set(att_src).at[1, :30].set(att_dst))
    wb = (jnp.zeros((4, 128), jnp.float32)
          .at[0, :30].set(b_gat)
          .at[1, :30].set(W1[:, 0])
          .at[2, :G].set(b2)
          .at[3, 0].set(b1[0]))
    w2t_pad = jnp.zeros((G, F), jnp.float32).at[:, :30].set(W2.T)
    bat_f = jnp.concatenate(
        [batch.astype(jnp.float32), jnp.full((NPAD - N,), 1e9, jnp.float32)])
    bat_pad = jnp.concatenate(
        [batch, jnp.full((NPAD - N,), 2 ** 30, jnp.int32)])
    bmin = bat_pad[0::BT]
    bmax = bat_pad[BT - 1::BT]

    ht, avec, srcr, dstr = pl.pallas_call(
        _k1_body,
        out_shape=[jax.ShapeDtypeStruct((F, NPAD), jnp.float32),
                   jax.ShapeDtypeStruct((8, NPAD), jnp.float32),
                   jax.ShapeDtypeStruct((NQ, EPQ), jnp.int32),
                   jax.ShapeDtypeStruct((NQ, EPQ), jnp.int32)],
    )(x, wgt_pad, att2, edge_index)

    s_part, o_part, _ = _run_sc(srcr, dstr, avec, ht)

    bpm, x1, x64 = pl.pallas_call(
        _k3_body,
        out_shape=[jax.ShapeDtypeStruct((F2, NPAD), jnp.bfloat16),
                   jax.ShapeDtypeStruct((NPAD,), jnp.float32),
                   jax.ShapeDtypeStruct((G, NPAD), jnp.float32)],
    )(o_part, s_part, avec, ht, w2t_pad, wb, bat_f)

    dup = pl.pallas_call(
        _k4_body,
        grid_spec=pltpu.PrefetchScalarGridSpec(
            num_scalar_prefetch=2,
            grid=(NB, NB),
            in_specs=[
                pl.BlockSpec((F2, BT), lambda i, j, bn, bx: (0, i)),
                pl.BlockSpec((F2, BT), lambda i, j, bn, bx: (0, j)),
            ],
            out_specs=pl.BlockSpec((BT,), lambda i, j, bn, bx: (i,)),
            scratch_shapes=[pltpu.VMEM((BT, BT), jnp.float32)],
        ),
        out_shape=jax.ShapeDtypeStruct((NPAD,), jnp.float32),
    )(bmin, bmax, bpm, bpm)

    gx = pl.pallas_call(
        _k5_body,
        out_shape=jax.ShapeDtypeStruct((G, G), jnp.float32),
    )(x1, x64, dup, bat_f)
    return gx


# speedup vs baseline: 49.0389x; 1.5493x over previous
import functools
import jax
import jax.numpy as jnp
from jax import lax
from jax.experimental import pallas as pl
from jax.experimental.pallas import tpu as pltpu
from jax.experimental.pallas import tpu_sc as plsc

N = 10000
NPAD = 10240
E = 160000
HID = 256
F = 32
G = 64
NW = 32
EPT = 5120
EPAD = NW * EPT
NQ = 4
EPQ = EPAD // NQ
CE = 4096
NCH = EPQ // CE
FPT = 4
BT3 = 256
NB3 = NPAD // BT3
F2 = 48


def _k1_body(x_ref, wgt_ref, att_ref, ei_ref, ht_ref, avec_ref,
             srcr_ref, dstr_ref):
    ER = E - 3 * EPQ
    for q in range(NQ - 1):
        srcr_ref[q, :] = ei_ref[0, pl.ds(q * EPQ, EPQ)]
        dstr_ref[q, :] = ei_ref[1, pl.ds(q * EPQ, EPQ)]
    srcr_ref[3, 0:ER] = ei_ref[0, pl.ds(3 * EPQ, ER)]
    srcr_ref[3, ER:EPQ] = jnp.zeros((EPQ - ER,), jnp.int32)
    dstr_ref[3, 0:ER] = ei_ref[1, pl.ds(3 * EPQ, ER)]
    dstr_ref[3, ER:EPQ] = jnp.full((EPQ - ER,), N, jnp.int32)
    ht = lax.dot_general(wgt_ref[...], x_ref[...],
                         (((1,), (1,)), ((), ())),
                         preferred_element_type=jnp.float32)
    ht_ref[:, 0:N] = ht
    ht_ref[:, N:NPAD] = jnp.zeros((F, NPAD - N), jnp.float32)
    a_s = jnp.sum(ht * att_ref[0, 0:F][:, None], axis=0)
    a_d = jnp.sum(ht * att_ref[1, 0:F][:, None], axis=0)
    e = a_s + a_d
    e = jnp.where(e >= 0, e, 0.2 * e)
    p_self = jnp.exp(e)
    zpad = jnp.zeros((NPAD - N,), jnp.float32)
    avec_ref[0, 0:N] = a_s
    avec_ref[0, N:NPAD] = zpad
    avec_ref[1, 0:N] = a_d
    avec_ref[1, N:NPAD] = jnp.full((NPAD - N,), -1e30, jnp.float32)
    avec_ref[2, 0:N] = p_self
    avec_ref[2, N:NPAD] = zpad


def _sc_edges(src_hbm, dst_hbm, avec_hbm, ht_hbm,
              s_out, o_out, p_out,
              as_v, ad_v, h2_v, h3_v, srcv, dstv, pv,
              s_loc, acc1, acc2, acc3,
              srcc, dstc, pc, srcc2, dstc2, pc2, sem0, sema, semb):
    c = lax.axis_index("c")
    s = lax.axis_index("s")
    wid = c * 16 + s
    wq = wid // 8
    wo = (wid % 8) * EPT
    pltpu.async_copy(avec_hbm.at[0], as_v, sem0)
    pltpu.async_copy(avec_hbm.at[1], ad_v, sem0)
    pltpu.async_copy(src_hbm.at[wq, pl.ds(wo, EPT)], srcv, sem0)
    pltpu.async_copy(dst_hbm.at[wq, pl.ds(wo, EPT)], dstv, sem0)

    zero16 = jnp.zeros((16,), jnp.float32)

    def zloop(i, _):
        for u in range(4):
            s_loc[pl.ds(i * 64 + u * 16, 16)] = zero16
        return 0
    lax.fori_loop(0, NPAD // 64, zloop, 0)

    pltpu.make_async_copy(avec_hbm.at[0], as_v, sem0).wait()
    pltpu.make_async_copy(avec_hbm.at[1], ad_v, sem0).wait()
    pltpu.make_async_copy(src_hbm.at[wq, pl.ds(wo, EPT)], srcv, sem0).wait()
    pltpu.make_async_copy(dst_hbm.at[wq, pl.ds(wo, EPT)], dstv, sem0).wait()

    @plsc.parallel_loop(0, EPT // 16, 1, unroll=8)
    def ploop(t):
        off = t * 16
        sv = srcv[pl.ds(off, 16)]
        dv = dstv[pl.ds(off, 16)]
        a1 = plsc.load_gather(as_v, [sv])
        a2 = plsc.load_gather(ad_v, [dv])
        e = a1 + a2
        e = jnp.where(e >= 0, e, 0.2 * e)
        pe = jnp.exp(e)
        pv[pl.ds(off, 16)] = pe
        plsc.addupdate_scatter(s_loc, [dv], pe)

    pltpu.sync_copy(pv, p_out.at[wq, pl.ds(wo, EPT)])
    pltpu.sync_copy(s_loc, s_out.at[wid])
    plsc.subcore_barrier()

    f0 = (wid % 8) * FPT
    pltpu.async_copy(ht_hbm.at[f0], as_v, sem0)
    pltpu.async_copy(ht_hbm.at[f0 + 1], ad_v, sem0)
    pltpu.async_copy(ht_hbm.at[f0 + 2], h2_v, sem0)
    pltpu.async_copy(ht_hbm.at[f0 + 3], h3_v, sem0)

    def zloop2(i, _):
        for u in range(2):
            o = i * 32 + u * 16
            s_loc[pl.ds(o, 16)] = zero16
            acc1[pl.ds(o, 16)] = zero16
            acc2[pl.ds(o, 16)] = zero16
            acc3[pl.ds(o, 16)] = zero16
        return 0
    lax.fori_loop(0, NPAD // 32, zloop2, 0)

    pltpu.make_async_copy(ht_hbm.at[f0], as_v, sem0).wait()
    pltpu.make_async_copy(ht_hbm.at[f0 + 1], ad_v, sem0).wait()
    pltpu.make_async_copy(ht_hbm.at[f0 + 2], h2_v, sem0).wait()
    pltpu.make_async_copy(ht_hbm.at[f0 + 3], h3_v, sem0).wait()

    def fire(k, sb, db, pb, sem):
        pltpu.async_copy(src_hbm.at[wq, pl.ds(k * CE, CE)], sb, sem)
        pltpu.async_copy(dst_hbm.at[wq, pl.ds(k * CE, CE)], db, sem)
        pltpu.async_copy(p_out.at[wq, pl.ds(k * CE, CE)], pb, sem)

    def drain(k, sb, db, pb, sem):
        pltpu.make_async_copy(src_hbm.at[wq, pl.ds(k * CE, CE)], sb, sem).wait()
        pltpu.make_async_copy(dst_hbm.at[wq, pl.ds(k * CE, CE)], db, sem).wait()
        pltpu.make_async_copy(p_out.at[wq, pl.ds(k * CE, CE)], pb, sem).wait()

    def compute(sb, db, pb):
        @plsc.parallel_loop(0, CE // 16, 1, unroll=8)
        def inner(i):
            off = i * 16
            sv = sb[pl.ds(off, 16)]
            dv = db[pl.ds(off, 16)]
            pe = pb[pl.ds(off, 16)]
            g0 = plsc.load_gather(as_v, [sv])
            plsc.addupdate_scatter(s_loc, [dv], g0 * pe)
            g1 = plsc.load_gather(ad_v, [sv])
            plsc.addupdate_scatter(acc1, [dv], g1 * pe)
            g2 = plsc.load_gather(h2_v, [sv])
            plsc.addupdate_scatter(acc2, [dv], g2 * pe)
            g3 = plsc.load_gather(h3_v, [sv])
            plsc.addupdate_scatter(acc3, [dv], g3 * pe)

    fire(0, srcc, dstc, pc, sema)

    def chunk2(g, _):
        k0 = g * 2
        fire(k0 + 1, srcc2, dstc2, pc2, semb)
        drain(k0, srcc, dstc, pc, sema)
        compute(srcc, dstc, pc)

        @pl.when(k0 + 2 < NCH)
        def _():
            fire(k0 + 2, srcc, dstc, pc, sema)
        drain(k0 + 1, srcc2, dstc2, pc2, semb)
        compute(srcc2, dstc2, pc2)
        return 0
    lax.fori_loop(0, NCH // 2, chunk2, 0)

    pltpu.sync_copy(s_loc, o_out.at[wq, f0])
    pltpu.sync_copy(acc1, o_out.at[wq, f0 + 1])
    pltpu.sync_copy(acc2, o_out.at[wq, f0 + 2])
    pltpu.sync_copy(acc3, o_out.at[wq, f0 + 3])


def _run_sc(srcr, dstr, avec, ht):
    mesh = plsc.VectorSubcoreMesh(core_axis_name="c", subcore_axis_name="s",
                                  num_cores=2, num_subcores=16)
    f = functools.partial(
        pl.kernel,
        out_type=[jax.ShapeDtypeStruct((NW, NPAD), jnp.float32),
                  jax.ShapeDtypeStruct((NQ, F, NPAD), jnp.float32),
                  jax.ShapeDtypeStruct((NQ, EPQ), jnp.float32)],
        mesh=mesh,
        compiler_params=pltpu.CompilerParams(needs_layout_passes=False),
        scratch_types=[
            pltpu.VMEM((NPAD,), jnp.float32),
            pltpu.VMEM((NPAD,), jnp.float32),
            pltpu.VMEM((NPAD,), jnp.float32),
            pltpu.VMEM((NPAD,), jnp.float32),
            pltpu.VMEM((EPT,), jnp.int32),
            pltpu.VMEM((EPT,), jnp.int32),
            pltpu.VMEM((EPT,), jnp.float32),
            pltpu.VMEM((NPAD,), jnp.float32),
            pltpu.VMEM((NPAD,), jnp.float32),
            pltpu.VMEM((NPAD,), jnp.float32),
            pltpu.VMEM((NPAD,), jnp.float32),
            pltpu.VMEM((CE,), jnp.int32),
            pltpu.VMEM((CE,), jnp.int32),
            pltpu.VMEM((CE,), jnp.float32),
            pltpu.VMEM((CE,), jnp.int32),
            pltpu.VMEM((CE,), jnp.int32),
            pltpu.VMEM((CE,), jnp.float32),
            pltpu.SemaphoreType.DMA,
            pltpu.SemaphoreType.DMA,
            pltpu.SemaphoreType.DMA,
        ],
    )(_sc_edges)
    return f(srcr, dstr, avec, ht)


def _k345_body(jlo_ref, opart_ref, spart_ref, avec_ref, ht_ref, w2t_ref,
               wb_ref, bat_ref, gx_ref, bpm_scr, dup_scr):
    num = (opart_ref[0] + opart_ref[1]) + (opart_ref[2] + opart_ref[3])
    s_edges = jnp.sum(spart_ref[...], axis=0)
    p_self = avec_ref[2, :]
    ht = ht_ref[...]
    s_tot = s_edges + p_self + 1e-16
    xc = (num + p_self[None, :] * ht) / s_tot[None, :]
    xc = xc + wb_ref[0, 0:F][:, None]
    sg = jnp.where(xc > 0, 1.0, -1.0)
    bat = bat_ref[...]
    bc = jnp.minimum(bat, 63.0)
    brows = []
    for k in range(6):
        q = jnp.floor(bc * 0.5)
        brows.append((2.0 * (bc - 2.0 * q) - 1.0)[None, :])
        bc = q
    zrows = jnp.zeros((F2 - F - 6, NPAD), jnp.float32)
    bpm_scr[...] = jnp.concatenate([sg] + brows + [zrows],
                                   axis=0).astype(jnp.bfloat16)

    x1 = jnp.sum(xc * wb_ref[1, 0:F][:, None], axis=0) + wb_ref[3, 0]
    x64 = lax.dot_general(w2t_ref[...], xc, (((1,), (0,)), ((), ())),
                          preferred_element_type=jnp.float32)
    x64 = x64 + wb_ref[2, 0:G][:, None]

    li = lax.broadcasted_iota(jnp.int32, (BT3, BT3), 0)
    lj = lax.broadcasted_iota(jnp.int32, (BT3, BT3), 1)
    dmask = jnp.where(lj < li, 0.0, -4096.0)
    for i in range(NB3):
        bi = bpm_scr[:, i * BT3:(i + 1) * BT3]
        simd = lax.dot_general(bi, bi, (((0,), (0,)), ((), ())),
                               preferred_element_type=jnp.float32)
        rm0 = jnp.max(simd + dmask, axis=1)
        d0 = jnp.where(rm0 > 37.0, 1.0, 0.0)

        def jbody(j, acc):
            bj = bpm_scr[:, pl.ds(j * BT3, BT3)]
            sim = lax.dot_general(bi, bj, (((0,), (0,)), ((), ())),
                                  preferred_element_type=jnp.float32)
            rm = jnp.max(sim, axis=1)
            return jnp.maximum(acc, jnp.where(rm > 37.0, 1.0, 0.0))
        d = lax.fori_loop(jlo_ref[i], i, jbody, d0)
        dup_scr[pl.ds(i * BT3, BT3)] = d

    pw = jnp.where(dup_scr[...] < 0.5, jnp.exp(x1), 0.0)
    gids = lax.broadcasted_iota(jnp.int32, (G, NPAD), 0).astype(jnp.float32)
    oht = jnp.where(gids == bat[None, :], 1.0, 0.0)
    wx = x64 * pw[None, :]
    numm = lax.dot_general(oht, wx, (((1,), (1,)), ((), ())),
                           preferred_element_type=jnp.float32)
    den = jnp.sum(oht * pw[None, :], axis=1, keepdims=True)
    gx_ref[...] = numm / (den + 1e-16)


def kernel(x, edge_index, batch, W_gat, att_src, att_dst, b_gat,
           W1, b1, W2, b2):
    wgt_pad = jnp.zeros((F, HID), jnp.float32).at[:30, :].set(W_gat.T)
    att2 = (jnp.zeros((2, F), jnp.float32)
            .at[0, :30].set(att_src).at[1, :30].set(att_dst))
    wb = (jnp.zeros((4, 128), jnp.float32)
          .at[0, :30].set(b_gat)
          .at[1, :30].set(W1[:, 0])
          .at[2, :G].set(b2)
          .at[3, 0].set(b1[0]))
    w2t_pad = jnp.zeros((G, F), jnp.float32).at[:, :30].set(W2.T)
    bat_f = jnp.concatenate(
        [batch.astype(jnp.float32), jnp.full((NPAD - N,), 1e9, jnp.float32)])
    bat_pad = jnp.concatenate(
        [batch, jnp.full((NPAD - N,), 2 ** 30, jnp.int32)])
    bmin = bat_pad[0::BT3]
    bmax = bat_pad[BT3 - 1::BT3]

    ht, avec, srcr, dstr = pl.pallas_call(
        _k1_body,
        out_shape=[jax.ShapeDtypeStruct((F, NPAD), jnp.float32),
                   jax.ShapeDtypeStruct((8, NPAD), jnp.float32),
                   jax.ShapeDtypeStruct((NQ, EPQ), jnp.int32),
                   jax.ShapeDtypeStruct((NQ, EPQ), jnp.int32)],
    )(x, wgt_pad, att2, edge_index)

    s_part, o_part, _ = _run_sc(srcr, dstr, avec, ht)

    jlo = jnp.searchsorted(bmax, bmin, side="left").astype(jnp.int32)

    gx = pl.pallas_call(
        _k345_body,
        in_specs=[pl.BlockSpec(memory_space=pltpu.MemorySpace.SMEM)]
        + [pl.BlockSpec()] * 7,
        scratch_shapes=[
            pltpu.VMEM((F2, NPAD), jnp.bfloat16),
            pltpu.VMEM((NPAD,), jnp.float32),
        ],
        out_shape=jax.ShapeDtypeStruct((G, G), jnp.float32),
    )(jlo, o_part, s_part, avec, ht, w2t_pad, wb, bat_f)
    return gx


# speedup vs baseline: 49.8277x; 1.0161x over previous
import functools
import jax
import jax.numpy as jnp
from jax import lax
from jax.experimental import pallas as pl
from jax.experimental.pallas import tpu as pltpu
from jax.experimental.pallas import tpu_sc as plsc

N = 10000
NPAD = 10240
E = 160000
HID = 256
F = 32
G = 64
NW = 32
EPT = 5120
EPAD = NW * EPT
NQ = 4
EPQ = EPAD // NQ
CE = 4096
NCH = EPQ // CE
FPT = 4
BT3 = 256
NB3 = NPAD // BT3
F2 = 48


def _k1_body(x_ref, wgt_ref, att_ref, ei_ref, ht_ref, avec_ref, pk_ref):
    ER = E - 3 * EPQ
    for q in range(NQ - 1):
        pk_ref[q, :] = (ei_ref[0, pl.ds(q * EPQ, EPQ)] * 16384
                        + ei_ref[1, pl.ds(q * EPQ, EPQ)])
    pk_ref[3, 0:ER] = (ei_ref[0, pl.ds(3 * EPQ, ER)] * 16384
                       + ei_ref[1, pl.ds(3 * EPQ, ER)])
    pk_ref[3, ER:EPQ] = jnp.full((EPQ - ER,), N, jnp.int32)
    ht = lax.dot_general(wgt_ref[...], x_ref[...],
                         (((1,), (1,)), ((), ())),
                         preferred_element_type=jnp.float32)
    ht_ref[:, 0:N] = ht
    ht_ref[:, N:NPAD] = jnp.zeros((F, NPAD - N), jnp.float32)
    a_s = jnp.sum(ht * att_ref[0, 0:F][:, None], axis=0)
    a_d = jnp.sum(ht * att_ref[1, 0:F][:, None], axis=0)
    e = a_s + a_d
    e = jnp.where(e >= 0, e, 0.2 * e)
    p_self = jnp.exp(e)
    zpad = jnp.zeros((NPAD - N,), jnp.float32)
    avec_ref[0, 0:N] = a_s
    avec_ref[0, N:NPAD] = zpad
    avec_ref[1, 0:N] = a_d
    avec_ref[1, N:NPAD] = jnp.full((NPAD - N,), -1e30, jnp.float32)
    avec_ref[2, 0:N] = p_self
    avec_ref[2, N:NPAD] = zpad


def _sc_edges(pk_hbm, avec_hbm, ht_hbm,
              s_out, o_out, p_out,
              as_v, ad_v, h2_v, h3_v, pkv, pv,
              s_loc, acc1, acc2, acc3,
              pkc, pc, pkc2, pc2, sem0, sema, semb):
    c = lax.axis_index("c")
    s = lax.axis_index("s")
    wid = c * 16 + s
    wq = wid // 8
    wo = (wid % 8) * EPT
    pltpu.async_copy(avec_hbm.at[0], as_v, sem0)
    pltpu.async_copy(avec_hbm.at[1], ad_v, sem0)
    pltpu.async_copy(pk_hbm.at[wq, pl.ds(wo, EPT)], pkv, sem0)

    zero16 = jnp.zeros((16,), jnp.float32)

    def zloop(i, _):
        for u in range(4):
            s_loc[pl.ds(i * 64 + u * 16, 16)] = zero16
        return 0
    lax.fori_loop(0, NPAD // 64, zloop, 0)

    pltpu.make_async_copy(avec_hbm.at[0], as_v, sem0).wait()
    pltpu.make_async_copy(avec_hbm.at[1], ad_v, sem0).wait()
    pltpu.make_async_copy(pk_hbm.at[wq, pl.ds(wo, EPT)], pkv, sem0).wait()

    @plsc.parallel_loop(0, EPT // 16, 1, unroll=8)
    def ploop(t):
        off = t * 16
        ev = pkv[pl.ds(off, 16)]
        sv = lax.shift_right_logical(ev, 14)
        dv = ev & 16383
        a1 = plsc.load_gather(as_v, [sv])
        a2 = plsc.load_gather(ad_v, [dv])
        e = a1 + a2
        e = jnp.where(e >= 0, e, 0.2 * e)
        pe = jnp.exp(e)
        pv[pl.ds(off, 16)] = pe
        plsc.addupdate_scatter(s_loc, [dv], pe)

    pltpu.sync_copy(pv, p_out.at[wq, pl.ds(wo, EPT)])
    pltpu.sync_copy(s_loc, s_out.at[wid])
    plsc.subcore_barrier()

    f0 = (wid % 8) * FPT
    pltpu.async_copy(ht_hbm.at[f0], as_v, sem0)
    pltpu.async_copy(ht_hbm.at[f0 + 1], ad_v, sem0)
    pltpu.async_copy(ht_hbm.at[f0 + 2], h2_v, sem0)
    pltpu.async_copy(ht_hbm.at[f0 + 3], h3_v, sem0)

    def zloop2(i, _):
        for u in range(2):
            o = i * 32 + u * 16
            s_loc[pl.ds(o, 16)] = zero16
            acc1[pl.ds(o, 16)] = zero16
            acc2[pl.ds(o, 16)] = zero16
            acc3[pl.ds(o, 16)] = zero16
        return 0
    lax.fori_loop(0, NPAD // 32, zloop2, 0)

    pltpu.make_async_copy(ht_hbm.at[f0], as_v, sem0).wait()
    pltpu.make_async_copy(ht_hbm.at[f0 + 1], ad_v, sem0).wait()
    pltpu.make_async_copy(ht_hbm.at[f0 + 2], h2_v, sem0).wait()
    pltpu.make_async_copy(ht_hbm.at[f0 + 3], h3_v, sem0).wait()

    def fire(k, eb, pb, sem):
        pltpu.async_copy(pk_hbm.at[wq, pl.ds(k * CE, CE)], eb, sem)
        pltpu.async_copy(p_out.at[wq, pl.ds(k * CE, CE)], pb, sem)

    def drain(k, eb, pb, sem):
        pltpu.make_async_copy(pk_hbm.at[wq, pl.ds(k * CE, CE)], eb, sem).wait()
        pltpu.make_async_copy(p_out.at[wq, pl.ds(k * CE, CE)], pb, sem).wait()

    def compute(eb, pb):
        @plsc.parallel_loop(0, CE // 16, 1, unroll=8)
        def inner(i):
            off = i * 16
            ev = eb[pl.ds(off, 16)]
            sv = lax.shift_right_logical(ev, 14)
            dv = ev & 16383
            pe = pb[pl.ds(off, 16)]
            g0 = plsc.load_gather(as_v, [sv])
            plsc.addupdate_scatter(s_loc, [dv], g0 * pe)
            g1 = plsc.load_gather(ad_v, [sv])
            plsc.addupdate_scatter(acc1, [dv], g1 * pe)
            g2 = plsc.load_gather(h2_v, [sv])
            plsc.addupdate_scatter(acc2, [dv], g2 * pe)
            g3 = plsc.load_gather(h3_v, [sv])
            plsc.addupdate_scatter(acc3, [dv], g3 * pe)

    fire(0, pkc, pc, sema)

    def chunk2(g, _):
        k0 = g * 2
        fire(k0 + 1, pkc2, pc2, semb)
        drain(k0, pkc, pc, sema)
        compute(pkc, pc)

        @pl.when(k0 + 2 < NCH)
        def _():
            fire(k0 + 2, pkc, pc, sema)
        drain(k0 + 1, pkc2, pc2, semb)
        compute(pkc2, pc2)
        return 0
    lax.fori_loop(0, NCH // 2, chunk2, 0)

    pltpu.sync_copy(s_loc, o_out.at[wq, f0])
    pltpu.sync_copy(acc1, o_out.at[wq, f0 + 1])
    pltpu.sync_copy(acc2, o_out.at[wq, f0 + 2])
    pltpu.sync_copy(acc3, o_out.at[wq, f0 + 3])


def _run_sc(pkr, avec, ht):
    mesh = plsc.VectorSubcoreMesh(core_axis_name="c", subcore_axis_name="s",
                                  num_cores=2, num_subcores=16)
    f = functools.partial(
        pl.kernel,
        out_type=[jax.ShapeDtypeStruct((NW, NPAD), jnp.float32),
                  jax.ShapeDtypeStruct((NQ, F, NPAD), jnp.float32),
                  jax.ShapeDtypeStruct((NQ, EPQ), jnp.float32)],
        mesh=mesh,
        compiler_params=pltpu.CompilerParams(needs_layout_passes=False),
        scratch_types=[
            pltpu.VMEM((NPAD,), jnp.float32),
            pltpu.VMEM((NPAD,), jnp.float32),
            pltpu.VMEM((NPAD,), jnp.float32),
            pltpu.VMEM((NPAD,), jnp.float32),
            pltpu.VMEM((EPT,), jnp.int32),
            pltpu.VMEM((EPT,), jnp.float32),
            pltpu.VMEM((NPAD,), jnp.float32),
            pltpu.VMEM((NPAD,), jnp.float32),
            pltpu.VMEM((NPAD,), jnp.float32),
            pltpu.VMEM((NPAD,), jnp.float32),
            pltpu.VMEM((CE,), jnp.int32),
            pltpu.VMEM((CE,), jnp.float32),
            pltpu.VMEM((CE,), jnp.int32),
            pltpu.VMEM((CE,), jnp.float32),
            pltpu.SemaphoreType.DMA,
            pltpu.SemaphoreType.DMA,
            pltpu.SemaphoreType.DMA,
        ],
    )(_sc_edges)
    return f(pkr, avec, ht)


def _k345_body(jlo_ref, opart_hbm, spart_hbm, avec_hbm, ht_hbm, w2t_ref,
               wb_ref, bat_ref, gx_ref, opart_ref, spart_ref, avec_ref,
               ht_ref, bpm_scr, dup_scr, sem0):
    for pair in ((opart_hbm, opart_ref), (spart_hbm, spart_ref),
                 (avec_hbm, avec_ref), (ht_hbm, ht_ref)):
        pltpu.async_copy(pair[0], pair[1], sem0)
    for pair in ((opart_hbm, opart_ref), (spart_hbm, spart_ref),
                 (avec_hbm, avec_ref), (ht_hbm, ht_ref)):
        pltpu.make_async_copy(pair[0], pair[1], sem0).wait()
    num = (opart_ref[0] + opart_ref[1]) + (opart_ref[2] + opart_ref[3])
    s_edges = jnp.sum(spart_ref[...], axis=0)
    p_self = avec_ref[2, :]
    ht = ht_ref[...]
    s_tot = s_edges + p_self + 1e-16
    xc = (num + p_self[None, :] * ht) / s_tot[None, :]
    xc = xc + wb_ref[0, 0:F][:, None]
    sg = jnp.where(xc > 0, 1.0, -1.0)
    bat = bat_ref[...]
    bc = jnp.minimum(bat, 63.0)
    brows = []
    for k in range(6):
        q = jnp.floor(bc * 0.5)
        brows.append((2.0 * (bc - 2.0 * q) - 1.0)[None, :])
        bc = q
    zrows = jnp.zeros((F2 - F - 6, NPAD), jnp.float32)
    bpm_scr[...] = jnp.concatenate([sg] + brows + [zrows],
                                   axis=0).astype(jnp.bfloat16)

    x1 = jnp.sum(xc * wb_ref[1, 0:F][:, None], axis=0) + wb_ref[3, 0]

    li = lax.broadcasted_iota(jnp.int32, (BT3, BT3), 0)
    lj = lax.broadcasted_iota(jnp.int32, (BT3, BT3), 1)
    dmask = jnp.where(lj < li, 0.0, -4096.0)
    for i in range(NB3):
        bi = bpm_scr[:, i * BT3:(i + 1) * BT3]
        simd = lax.dot_general(bi, bi, (((0,), (0,)), ((), ())),
                               preferred_element_type=jnp.float32)
        rm0 = jnp.max(simd + dmask, axis=1)
        d0 = jnp.where(rm0 > 37.0, 1.0, 0.0)

        def jbody(j, acc):
            bj = bpm_scr[:, pl.ds(j * BT3, BT3)]
            sim = lax.dot_general(bi, bj, (((0,), (0,)), ((), ())),
                                  preferred_element_type=jnp.float32)
            rm = jnp.max(sim, axis=1)
            return jnp.maximum(acc, jnp.where(rm > 37.0, 1.0, 0.0))
        d = lax.fori_loop(jlo_ref[i], i, jbody, d0)
        dup_scr[pl.ds(i * BT3, BT3)] = d

    pw = jnp.where(dup_scr[...] < 0.5, jnp.exp(x1), 0.0)
    gids = lax.broadcasted_iota(jnp.int32, (G, NPAD), 0).astype(jnp.float32)
    wxo = jnp.where(gids == bat[None, :], pw[None, :], 0.0)
    xcd = jnp.concatenate(
        [xc[0:30], jnp.ones((1, NPAD), jnp.float32), xc[31:32]], axis=0)
    m = lax.dot_general(wxo, xcd, (((1,), (1,)), ((), ())),
                        preferred_element_type=jnp.float32)
    numm = lax.dot_general(m, w2t_ref[...], (((1,), (1,)), ((), ())),
                           preferred_element_type=jnp.float32)
    den = m[:, 30:31]
    numm = numm + den * wb_ref[2, 0:G][None, :]
    gx_ref[...] = numm / (den + 1e-16)


def kernel(x, edge_index, batch, W_gat, att_src, att_dst, b_gat,
           W1, b1, W2, b2):
    wgt_pad = jnp.zeros((F, HID), jnp.float32).at[:30, :].set(W_gat.T)
    att2 = (jnp.zeros((2, F), jnp.float32)
            .at[0, :30].set(att_src).at[1, :30].set(att_dst))
    wb = (jnp.zeros((4, 128), jnp.float32)
          .at[0, :30].set(b_gat)
          .at[1, :30].set(W1[:, 0])
          .at[2, :G].set(b2)
          .at[3, 0].set(b1[0]))
    w2t_pad = jnp.zeros((G, F), jnp.float32).at[:, :30].set(W2.T)
    bat_f = jnp.concatenate(
        [batch.astype(jnp.float32), jnp.full((NPAD - N,), 1e9, jnp.float32)])
    bat_pad = jnp.concatenate(
        [batch, jnp.full((NPAD - N,), 2 ** 30, jnp.int32)])
    bmin = bat_pad[0::BT3]
    bmax = bat_pad[BT3 - 1::BT3]

    ht, avec, pkr = pl.pallas_call(
        _k1_body,
        out_shape=[jax.ShapeDtypeStruct((F, NPAD), jnp.float32),
                   jax.ShapeDtypeStruct((8, NPAD), jnp.float32),
                   jax.ShapeDtypeStruct((NQ, EPQ), jnp.int32)],
    )(x, wgt_pad, att2, edge_index)

    s_part, o_part, _ = _run_sc(pkr, avec, ht)

    jlo = jnp.searchsorted(bmax, bmin, side="left").astype(jnp.int32)

    gx = pl.pallas_call(
        _k345_body,
        in_specs=[pl.BlockSpec(memory_space=pltpu.MemorySpace.SMEM)]
        + [pl.BlockSpec(memory_space=pl.ANY)] * 4
        + [pl.BlockSpec()] * 3,
        scratch_shapes=[
            pltpu.VMEM((NQ, F, NPAD), jnp.float32),
            pltpu.VMEM((NW, NPAD), jnp.float32),
            pltpu.VMEM((8, NPAD), jnp.float32),
            pltpu.VMEM((F, NPAD), jnp.float32),
            pltpu.VMEM((F2, NPAD), jnp.bfloat16),
            pltpu.VMEM((NPAD,), jnp.float32),
            pltpu.SemaphoreType.DMA,
        ],
        out_shape=jax.ShapeDtypeStruct((G, G), jnp.float32),
    )(jlo, o_part, s_part, avec, ht, w2t_pad, wb, bat_f)
    return gx
